# Initial kernel scaffold; baseline (speedup 1.0000x reference)
import jax
import jax.numpy as jnp
from jax.experimental import pallas as pl


def kernel(x1, x2, edge_index_v1, edge_index_v2, prelu_a, W_w, W_b, W2_w, W2_b, l1_lw, l1_lb, l1_rw, l2_lw, l2_lb, l2_rw, l3_lw, l3_lb, l3_rw):
    raise NotImplementedError("write your pallas kernel here")


# speedup vs baseline: 2.6659x; 2.6659x over previous
import functools

import jax
import jax.numpy as jnp
from jax import lax
from jax.experimental import pallas as pl
from jax.experimental.pallas import tpu as pltpu
from jax.experimental.pallas import tpu_sc as plsc

NC = 2
NS = 16
LN = 16

N = 10000
NPAD = 10240
E = 320000
D = 128
H = 512

G = 128


@functools.cache
def _segsum_kernel(F, CHUNK, NCHUNK, with_count, n_rows):
    CPT = CHUNK // NS
    KPC = NCHUNK // NC
    EPT = E // NS
    EB = 2000
    NB = EPT // EB
    GRP = EB // LN
    FV = F // LN

    assert CHUNK % NS == 0 and NCHUNK % NC == 0 and NCHUNK * CHUNK == NPAD
    assert EPT % EB == 0 and EB % LN == 0 and CPT % LN == 0

    mesh = plsc.VectorSubcoreMesh(core_axis_name="c", subcore_axis_name="s")

    out_type = [jax.ShapeDtypeStruct((NPAD, F), jnp.float32)]
    if with_count:
        out_type.append(jax.ShapeDtypeStruct((NPAD, LN), jnp.float32))

    scratch = [
        pltpu.VMEM((EB,), jnp.int32),
        pltpu.VMEM((EB,), jnp.int32),
        pltpu.VMEM((G,), jnp.int32),
        pltpu.VMEM((G,), jnp.int32),
        pltpu.VMEM((G, F), jnp.float32),
        pltpu.VMEM((LN, F), jnp.float32),
        pltpu.VMEM_SHARED((CHUNK + LN, F), jnp.float32),
        pltpu.SemaphoreType.DMA,
    ]
    if with_count:
        scratch += [
            pltpu.VMEM((G, LN), jnp.float32),
            pltpu.VMEM((LN, LN), jnp.float32),
            pltpu.VMEM_SHARED((CHUNK + LN, LN), jnp.float32),
        ]

    def body(src_h, dst_h, x_h, out_h, *rest):
        if with_count:
            (cnt_h, src_v, dst_v, gsrc, gdst, rows, zbuf, acc, sem,
             ones, zbufc, cacc) = rest
        else:
            (src_v, dst_v, gsrc, gdst, rows, zbuf, acc, sem) = rest

        c = lax.axis_index("c")
        s = lax.axis_index("s")
        zero16 = jnp.zeros((LN,), jnp.float32)
        izero = jnp.zeros((LN,), jnp.int32)
        pad_src = izero + s
        pad_dst = izero + (CHUNK + s)

        def zb(k, _):
            zbuf[k // FV, pl.ds((k % FV) * LN, LN)] = zero16
            return 0
        lax.fori_loop(0, LN * FV, zb, 0)
        if with_count:
            one16 = zero16 + 1.0

            def ob(k, _):
                ones[k, :] = one16
                return 0
            lax.fori_loop(0, G, ob, 0)

            def zc(k, _):
                zbufc[k, :] = zero16
                return 0
            lax.fori_loop(0, LN, zc, 0)

        def refill_pad():
            for j in range(G // LN):
                gsrc[pl.ds(j * LN, LN)] = pad_src
                gdst[pl.ds(j * LN, LN)] = pad_dst

        def flush():
            pltpu.async_copy(x_h.at[gsrc], rows, sem).wait()
            pltpu.sync_copy(rows, acc.at[gdst], add=True)
            if with_count:
                pltpu.sync_copy(ones, cacc.at[gdst], add=True)
            refill_pad()

        for k in range(KPC):
            chunk_id = k * NC + c
            lo = chunk_id * CHUNK

            for i in range(CPT // LN):
                pltpu.sync_copy(zbuf, acc.at[pl.ds(s * CPT + i * LN, LN)])
            if with_count:
                for i in range(CPT // LN):
                    pltpu.sync_copy(zbufc, cacc.at[pl.ds(s * CPT + i * LN, LN)])
            @pl.when(s == 0)
            def _():
                pltpu.sync_copy(zbuf, acc.at[pl.ds(CHUNK, LN)])
                if with_count:
                    pltpu.sync_copy(zbufc, cacc.at[pl.ds(CHUNK, LN)])
            refill_pad()
            plsc.subcore_barrier()

            ebase = s * EPT

            def grp(gi, pend):
                off = gi * LN
                sv = src_v[pl.ds(off, LN)]
                dv = dst_v[pl.ds(off, LN)]
                m = (dv >= lo) & (dv < lo + CHUNK)
                mi = m.astype(jnp.int32)
                pos = pend + plsc.cumsum(mi) - 1
                plsc.store_scatter(gsrc, [pos], sv, mask=m)
                plsc.store_scatter(gdst, [pos], dv - lo, mask=m)
                pend = pend + jnp.sum(mi)
                full = pend > (G - LN)

                @pl.when(full)
                def _():
                    flush()
                return jnp.where(full, 0, pend)

            def blk(b, pend):
                pltpu.sync_copy(src_h.at[pl.ds(ebase + b * EB, EB)], src_v)
                pltpu.sync_copy(dst_h.at[pl.ds(ebase + b * EB, EB)], dst_v)
                return lax.fori_loop(0, GRP, grp, pend)

            lax.fori_loop(0, NB, blk, jnp.int32(0))
            flush()
            plsc.subcore_barrier()

            pltpu.sync_copy(acc.at[pl.ds(s * CPT, CPT)],
                            out_h.at[pl.ds(lo + s * CPT, CPT)])
            if with_count:
                pltpu.sync_copy(cacc.at[pl.ds(s * CPT, CPT)],
                                cnt_h.at[pl.ds(lo + s * CPT, CPT)])

    return pl.kernel(body, out_type=tuple(out_type), mesh=mesh,
                     scratch_types=scratch,
                     compiler_params=pltpu.CompilerParams(
                         needs_layout_passes=False,
                         use_tc_tiling_on_sc=False))


_CT = (((1,), (1,)), ((), ()))


def _dot(a, b):
    return lax.dot_general(a, b, _CT, preferred_element_type=jnp.float32)


@functools.cache
def _layer1_kernel(F_in):
    BR = 256

    def body(s_ref, c_ref, x_ref, a_ref, lw_ref, lb_ref, rw_ref,
             ww_ref, wb_ref, w2_ref, w2b_ref, in2_ref, r3_ref):
        cnt = jnp.maximum(c_ref[:, 0:1], 1.0)
        mean = s_ref[...] / cnt
        t = _dot(mean, lw_ref[...]) + lb_ref[...] + _dot(x_ref[...], rw_ref[...])
        a = a_ref[...]
        h1 = jnp.where(t >= 0, t, a * t)
        in2_ref[...] = h1 + _dot(x_ref[...], ww_ref[...]) + wb_ref[...]
        r3_ref[...] = h1 + _dot(x_ref[...], w2_ref[...]) + w2b_ref[...]

    grid = (NPAD // BR,)
    row = lambda i: (i, 0)
    fix = lambda i: (0, 0)
    return pl.pallas_call(
        body,
        grid=grid,
        in_specs=[
            pl.BlockSpec((BR, F_in), row),
            pl.BlockSpec((BR, LN), row),
            pl.BlockSpec((BR, F_in), row),
            pl.BlockSpec((1, 1), fix),
            pl.BlockSpec((H, F_in), fix),
            pl.BlockSpec((1, H), fix),
            pl.BlockSpec((H, F_in), fix),
            pl.BlockSpec((H, F_in), fix),
            pl.BlockSpec((1, H), fix),
            pl.BlockSpec((H, F_in), fix),
            pl.BlockSpec((1, H), fix),
        ],
        out_specs=[pl.BlockSpec((BR, H), row), pl.BlockSpec((BR, H), row)],
        out_shape=[jax.ShapeDtypeStruct((NPAD, H), jnp.float32)] * 2,
    )


@functools.cache
def _layer2_kernel():
    BR = 256

    def body(s_ref, c_ref, in_ref, r3_ref, a_ref, lw_ref, lb_ref, rw_ref,
             out_ref):
        cnt = jnp.maximum(c_ref[:, 0:1], 1.0)
        mean = s_ref[...] / cnt
        t = _dot(mean, lw_ref[...]) + lb_ref[...] + _dot(in_ref[...], rw_ref[...])
        a = a_ref[...]
        h2 = jnp.where(t >= 0, t, a * t)
        out_ref[...] = h2 + r3_ref[...]

    grid = (NPAD // 256,)
    row = lambda i: (i, 0)
    fix = lambda i: (0, 0)
    return pl.pallas_call(
        body,
        grid=grid,
        in_specs=[
            pl.BlockSpec((BR, H), row),
            pl.BlockSpec((BR, LN), row),
            pl.BlockSpec((BR, H), row),
            pl.BlockSpec((BR, H), row),
            pl.BlockSpec((1, 1), fix),
            pl.BlockSpec((H, H), fix),
            pl.BlockSpec((1, H), fix),
            pl.BlockSpec((H, H), fix),
        ],
        out_specs=pl.BlockSpec((BR, H), row),
        out_shape=jax.ShapeDtypeStruct((NPAD, H), jnp.float32),
    )


@functools.cache
def _layer3_kernel():
    BR = 256

    def body(s_ref, c_ref, in_ref, a_ref, lw_ref, lb_ref, rw_ref, out_ref):
        cnt = jnp.maximum(c_ref[:, 0:1], 1.0)
        mean = s_ref[...] / cnt
        t = _dot(mean, lw_ref[...]) + lb_ref[...] + _dot(in_ref[...], rw_ref[...])
        a = a_ref[...]
        out_ref[...] = jnp.where(t >= 0, t, a * t)

    grid = (NPAD // 256,)
    row = lambda i: (i, 0)
    fix = lambda i: (0, 0)
    return pl.pallas_call(
        body,
        grid=grid,
        in_specs=[
            pl.BlockSpec((BR, H), row),
            pl.BlockSpec((BR, LN), row),
            pl.BlockSpec((BR, H), row),
            pl.BlockSpec((1, 1), fix),
            pl.BlockSpec((H, H), fix),
            pl.BlockSpec((1, H), fix),
            pl.BlockSpec((H, H), fix),
        ],
        out_specs=pl.BlockSpec((BR, H), row),
        out_shape=jax.ShapeDtypeStruct((NPAD, H), jnp.float32),
    )


def _encode(x, ei, a2, lw1, lb1, rw1, lw2, lb2, rw2, lw3, lb3, rw3,
            Ww, Wb, W2w, W2b):
    src = ei[0]
    dst = ei[1]
    seg128 = _segsum_kernel(D, 2560, 4, True, N)
    seg512 = _segsum_kernel(H, 1280, 8, False, N)

    s1, c16 = seg128(src, dst, x)
    xp = jnp.pad(x, ((0, NPAD - N), (0, 0)))
    in2, r3 = _layer1_kernel(D)(s1, c16, xp, a2, lw1, lb1, rw1,
                                Ww, Wb, W2w, W2b)
    (s2,) = seg512(src, dst, in2)
    in3 = _layer2_kernel()(s2, c16, in2, r3, a2, lw2, lb2, rw2)
    (s3,) = seg512(src, dst, in3)
    h3 = _layer3_kernel()(s3, c16, in3, a2, lw3, lb3, rw3)
    return h3[:N]


def kernel(x1, x2, edge_index_v1, edge_index_v2, prelu_a, W_w, W_b, W2_w,
           W2_b, l1_lw, l1_lb, l1_rw, l2_lw, l2_lb, l2_rw, l3_lw, l3_lb,
           l3_rw):
    a2 = prelu_a.reshape(1, 1)
    lb1 = l1_lb.reshape(1, H)
    lb2 = l2_lb.reshape(1, H)
    lb3 = l3_lb.reshape(1, H)
    Wb = W_b.reshape(1, H)
    W2b = W2_b.reshape(1, H)
    args = (a2, l1_lw, lb1, l1_rw, l2_lw, lb2, l2_rw, l3_lw, lb3, l3_rw,
            W_w, Wb, W2_w, W2b)
    out1 = _encode(x1, edge_index_v1, *args)
    out1, x2b, ei2b = lax.optimization_barrier((out1, x2, edge_index_v2))
    out2 = _encode(x2b, ei2b, *args)
    return (out1, out2)


# speedup vs baseline: 2.7620x; 1.0361x over previous
import functools

import jax
import jax.numpy as jnp
from jax import lax
from jax.experimental import pallas as pl
from jax.experimental.pallas import tpu as pltpu
from jax.experimental.pallas import tpu_sc as plsc

NC = 2
NS = 16
LN = 16

N = 10000
NPAD = 10240
E = 320000
D = 128
H = 512

G = 128


@functools.cache
def _segsum_kernel(F, CHUNK, NCHUNK, with_count, n_rows):
    CPT = CHUNK // NS
    KPC = NCHUNK // NC
    EPT = E // NS
    EB = 2000
    NB = EPT // EB
    GRP = EB // LN
    FV = F // LN

    assert CHUNK % NS == 0 and NCHUNK % NC == 0 and NCHUNK * CHUNK == NPAD
    assert EPT % EB == 0 and EB % LN == 0 and CPT % LN == 0

    mesh = plsc.VectorSubcoreMesh(core_axis_name="c", subcore_axis_name="s")

    out_type = [jax.ShapeDtypeStruct((NPAD, F), jnp.float32)]
    if with_count:
        out_type.append(jax.ShapeDtypeStruct((NPAD, LN), jnp.float32))

    scratch = [
        pltpu.VMEM((EB,), jnp.int32),
        pltpu.VMEM((EB,), jnp.int32),
        pltpu.VMEM((G,), jnp.int32),
        pltpu.VMEM((G,), jnp.int32),
        pltpu.VMEM((G, F), jnp.float32),
        pltpu.VMEM((LN, F), jnp.float32),
        pltpu.VMEM_SHARED((CHUNK + LN, F), jnp.float32),
        pltpu.SemaphoreType.DMA,
    ]
    if with_count:
        scratch += [
            pltpu.VMEM((G, LN), jnp.float32),
            pltpu.VMEM((LN, LN), jnp.float32),
            pltpu.VMEM_SHARED((CHUNK + LN, LN), jnp.float32),
        ]

    def body(src_h, dst_h, x_h, out_h, *rest):
        if with_count:
            (cnt_h, src_v, dst_v, gsrc, gdst, rows, zbuf, acc, sem,
             ones, zbufc, cacc) = rest
        else:
            (src_v, dst_v, gsrc, gdst, rows, zbuf, acc, sem) = rest

        c = lax.axis_index("c")
        s = lax.axis_index("s")
        zero16 = jnp.zeros((LN,), jnp.float32)
        izero = jnp.zeros((LN,), jnp.int32)
        pad_src = izero + s
        pad_dst = izero + (CHUNK + s)

        def zb(k, _):
            zbuf[k // FV, pl.ds((k % FV) * LN, LN)] = zero16
            return 0
        lax.fori_loop(0, LN * FV, zb, 0)
        if with_count:
            one16 = zero16 + 1.0

            def ob(k, _):
                ones[k, :] = one16
                return 0
            lax.fori_loop(0, G, ob, 0)

            def zc(k, _):
                zbufc[k, :] = zero16
                return 0
            lax.fori_loop(0, LN, zc, 0)

        def refill_pad():
            for j in range(G // LN):
                gsrc[pl.ds(j * LN, LN)] = pad_src
                gdst[pl.ds(j * LN, LN)] = pad_dst

        def flush():
            pltpu.async_copy(x_h.at[gsrc], rows, sem).wait()
            pltpu.sync_copy(rows, acc.at[gdst], add=True)
            if with_count:
                pltpu.sync_copy(ones, cacc.at[gdst], add=True)
            refill_pad()

        for k in range(KPC):
            chunk_id = k * NC + c
            lo = chunk_id * CHUNK

            for i in range(CPT // LN):
                pltpu.sync_copy(zbuf, acc.at[pl.ds(s * CPT + i * LN, LN)])
            if with_count:
                for i in range(CPT // LN):
                    pltpu.sync_copy(zbufc, cacc.at[pl.ds(s * CPT + i * LN, LN)])
            @pl.when(s == 0)
            def _():
                pltpu.sync_copy(zbuf, acc.at[pl.ds(CHUNK, LN)])
                if with_count:
                    pltpu.sync_copy(zbufc, cacc.at[pl.ds(CHUNK, LN)])
            refill_pad()
            plsc.subcore_barrier()

            ebase = s * EPT

            def grp(gi, pend):
                off = gi * LN
                sv = src_v[pl.ds(off, LN)]
                dv = dst_v[pl.ds(off, LN)]
                m = (dv >= lo) & (dv < lo + CHUNK)
                mi = m.astype(jnp.int32)
                pos = pend + plsc.cumsum(mi) - 1
                plsc.store_scatter(gsrc, [pos], sv, mask=m)
                plsc.store_scatter(gdst, [pos], dv - lo, mask=m)
                pend = pend + jnp.sum(mi)
                full = pend > (G - LN)

                @pl.when(full)
                def _():
                    flush()
                return jnp.where(full, 0, pend)

            def blk(b, pend):
                pltpu.sync_copy(src_h.at[pl.ds(ebase + b * EB, EB)], src_v)
                pltpu.sync_copy(dst_h.at[pl.ds(ebase + b * EB, EB)], dst_v)
                return lax.fori_loop(0, GRP, grp, pend)

            lax.fori_loop(0, NB, blk, jnp.int32(0))
            flush()
            plsc.subcore_barrier()

            pltpu.sync_copy(acc.at[pl.ds(s * CPT, CPT)],
                            out_h.at[pl.ds(lo + s * CPT, CPT)])
            if with_count:
                pltpu.sync_copy(cacc.at[pl.ds(s * CPT, CPT)],
                                cnt_h.at[pl.ds(lo + s * CPT, CPT)])

    return pl.kernel(body, out_type=tuple(out_type), mesh=mesh,
                     scratch_types=scratch,
                     compiler_params=pltpu.CompilerParams(
                         needs_layout_passes=False,
                         use_tc_tiling_on_sc=False))


_CT = (((1,), (1,)), ((), ()))


def _dot(a, b):
    return lax.dot_general(a, b, _CT, preferred_element_type=jnp.float32)


@functools.cache
def _layer1_kernel(F_in):
    BR = 256

    def body(s_ref, c_ref, x_ref, a_ref, lw_ref, lb_ref, rw_ref,
             ww_ref, wb_ref, w2_ref, w2b_ref, in2_ref, r3_ref):
        cnt = jnp.maximum(c_ref[:, 0:1], 1.0)
        mean = s_ref[...] / cnt
        t = _dot(mean, lw_ref[...]) + lb_ref[...] + _dot(x_ref[...], rw_ref[...])
        a = a_ref[...]
        h1 = jnp.where(t >= 0, t, a * t)
        in2_ref[...] = h1 + _dot(x_ref[...], ww_ref[...]) + wb_ref[...]
        r3_ref[...] = h1 + _dot(x_ref[...], w2_ref[...]) + w2b_ref[...]

    grid = (NPAD // BR,)
    row = lambda i: (i, 0)
    fix = lambda i: (0, 0)
    return pl.pallas_call(
        body,
        grid=grid,
        in_specs=[
            pl.BlockSpec((BR, F_in), row),
            pl.BlockSpec((BR, LN), row),
            pl.BlockSpec((BR, F_in), row),
            pl.BlockSpec((1, 1), fix),
            pl.BlockSpec((H, F_in), fix),
            pl.BlockSpec((1, H), fix),
            pl.BlockSpec((H, F_in), fix),
            pl.BlockSpec((H, F_in), fix),
            pl.BlockSpec((1, H), fix),
            pl.BlockSpec((H, F_in), fix),
            pl.BlockSpec((1, H), fix),
        ],
        out_specs=[pl.BlockSpec((BR, H), row), pl.BlockSpec((BR, H), row)],
        out_shape=[jax.ShapeDtypeStruct((NPAD, H), jnp.float32)] * 2,
    )


@functools.cache
def _layer2_kernel():
    BR = 256

    def body(s_ref, c_ref, in_ref, r3_ref, a_ref, lw_ref, lb_ref, rw_ref,
             out_ref):
        cnt = jnp.maximum(c_ref[:, 0:1], 1.0)
        mean = s_ref[...] / cnt
        t = _dot(mean, lw_ref[...]) + lb_ref[...] + _dot(in_ref[...], rw_ref[...])
        a = a_ref[...]
        h2 = jnp.where(t >= 0, t, a * t)
        out_ref[...] = h2 + r3_ref[...]

    grid = (NPAD // 256,)
    row = lambda i: (i, 0)
    fix = lambda i: (0, 0)
    return pl.pallas_call(
        body,
        grid=grid,
        in_specs=[
            pl.BlockSpec((BR, H), row),
            pl.BlockSpec((BR, LN), row),
            pl.BlockSpec((BR, H), row),
            pl.BlockSpec((BR, H), row),
            pl.BlockSpec((1, 1), fix),
            pl.BlockSpec((H, H), fix),
            pl.BlockSpec((1, H), fix),
            pl.BlockSpec((H, H), fix),
        ],
        out_specs=pl.BlockSpec((BR, H), row),
        out_shape=jax.ShapeDtypeStruct((NPAD, H), jnp.float32),
    )


@functools.cache
def _layer3_kernel():
    BR = 256

    def body(s_ref, c_ref, in_ref, a_ref, lw_ref, lb_ref, rw_ref, out_ref):
        cnt = jnp.maximum(c_ref[:, 0:1], 1.0)
        mean = s_ref[...] / cnt
        t = _dot(mean, lw_ref[...]) + lb_ref[...] + _dot(in_ref[...], rw_ref[...])
        a = a_ref[...]
        out_ref[...] = jnp.where(t >= 0, t, a * t)

    grid = (NPAD // 256,)
    row = lambda i: (i, 0)
    fix = lambda i: (0, 0)
    return pl.pallas_call(
        body,
        grid=grid,
        in_specs=[
            pl.BlockSpec((BR, H), row),
            pl.BlockSpec((BR, LN), row),
            pl.BlockSpec((BR, H), row),
            pl.BlockSpec((1, 1), fix),
            pl.BlockSpec((H, H), fix),
            pl.BlockSpec((1, H), fix),
            pl.BlockSpec((H, H), fix),
        ],
        out_specs=pl.BlockSpec((BR, H), row),
        out_shape=jax.ShapeDtypeStruct((NPAD, H), jnp.float32),
    )


def _encode(x, ei, a2, lw1, lb1, rw1, lw2, lb2, rw2, lw3, lb3, rw3,
            Ww, Wb, W2w, W2b):
    src = ei[0]
    dst = ei[1]
    seg128 = _segsum_kernel(D, 2560, 4, True, N)
    seg512 = _segsum_kernel(H, 1280, 8, False, N)

    s1, c16 = seg128(src, dst, x)
    xp = jnp.pad(x, ((0, NPAD - N), (0, 0)))
    in2, r3 = _layer1_kernel(D)(s1, c16, xp, a2, lw1, lb1, rw1,
                                Ww, Wb, W2w, W2b)
    (s2,) = seg512(src, dst, in2)
    in3 = _layer2_kernel()(s2, c16, in2, r3, a2, lw2, lb2, rw2)
    (s3,) = seg512(src, dst, in3)
    h3 = _layer3_kernel()(s3, c16, in3, a2, lw3, lb3, rw3)
    return h3[:N]


def kernel(x1, x2, edge_index_v1, edge_index_v2, prelu_a, W_w, W_b, W2_w,
           W2_b, l1_lw, l1_lb, l1_rw, l2_lw, l2_lb, l2_rw, l3_lw, l3_lb,
           l3_rw):
    a2 = prelu_a.reshape(1, 1)
    lb1 = l1_lb.reshape(1, H)
    lb2 = l2_lb.reshape(1, H)
    lb3 = l3_lb.reshape(1, H)
    Wb = W_b.reshape(1, H)
    W2b = W2_b.reshape(1, H)
    args = (a2, l1_lw, lb1, l1_rw, l2_lw, lb2, l2_rw, l3_lw, lb3, l3_rw,
            W_w, Wb, W2_w, W2b)
    out1 = _encode(x1, edge_index_v1, *args)
    out2 = _encode(x2, edge_index_v2, *args)
    return (out1, out2)


# speedup vs baseline: 3.7193x; 1.3466x over previous
import functools

import jax
import jax.numpy as jnp
from jax import lax
from jax.experimental import pallas as pl
from jax.experimental.pallas import tpu as pltpu
from jax.experimental.pallas import tpu_sc as plsc

NC = 2
NS = 16
LN = 16

N = 10000
NPAD = 10240
E = 320000
D = 128
H = 512

G = 128


@functools.cache
def _segsum_kernel(F, CHUNK, NCHUNK, with_count, n_rows, dtype=jnp.float32):
    CPT = CHUNK // NS
    KPC = NCHUNK // NC
    EPT = E // NS
    EB = 2000
    NB = EPT // EB
    GRP = EB // LN
    FV = F // LN

    assert CHUNK % NS == 0 and NCHUNK % NC == 0 and NCHUNK * CHUNK == NPAD
    assert EPT % EB == 0 and EB % LN == 0 and CPT % LN == 0

    mesh = plsc.VectorSubcoreMesh(core_axis_name="c", subcore_axis_name="s")

    out_type = [jax.ShapeDtypeStruct((NPAD, F), dtype)]
    if with_count:
        out_type.append(jax.ShapeDtypeStruct((NPAD, LN), jnp.float32))

    scratch = [
        pltpu.VMEM((EB,), jnp.int32),
        pltpu.VMEM((EB,), jnp.int32),
        pltpu.VMEM((G,), jnp.int32),
        pltpu.VMEM((G,), jnp.int32),
        pltpu.VMEM((G, F), dtype),
        pltpu.VMEM((LN, F), dtype),
        pltpu.VMEM_SHARED((CHUNK + LN, F), dtype),
        pltpu.SemaphoreType.DMA,
    ]
    if with_count:
        scratch += [
            pltpu.VMEM((G, LN), jnp.float32),
            pltpu.VMEM((LN, LN), jnp.float32),
            pltpu.VMEM_SHARED((CHUNK + LN, LN), jnp.float32),
        ]

    def body(src_h, dst_h, x_h, out_h, *rest):
        if with_count:
            (cnt_h, src_v, dst_v, gsrc, gdst, rows, zbuf, acc, sem,
             ones, zbufc, cacc) = rest
        else:
            (src_v, dst_v, gsrc, gdst, rows, zbuf, acc, sem) = rest

        c = lax.axis_index("c")
        s = lax.axis_index("s")
        zero16 = jnp.zeros((LN,), jnp.float32)
        izero = jnp.zeros((LN,), jnp.int32)
        pad_src = izero + s
        pad_dst = izero + (CHUNK + s)

        ZW = LN if dtype == jnp.float32 else 2 * LN
        zvec = jnp.zeros((ZW,), dtype)

        def zb(k, _):
            zbuf[k // (F // ZW), pl.ds((k % (F // ZW)) * ZW, ZW)] = zvec
            return 0
        lax.fori_loop(0, LN * (F // ZW), zb, 0)
        if with_count:
            one16 = zero16 + 1.0

            def ob(k, _):
                ones[k, :] = one16
                return 0
            lax.fori_loop(0, G, ob, 0)

            def zc(k, _):
                zbufc[k, :] = zero16
                return 0
            lax.fori_loop(0, LN, zc, 0)

        def refill_pad():
            for j in range(G // LN):
                gsrc[pl.ds(j * LN, LN)] = pad_src
                gdst[pl.ds(j * LN, LN)] = pad_dst

        def flush():
            pltpu.async_copy(x_h.at[gsrc], rows, sem).wait()
            pltpu.sync_copy(rows, acc.at[gdst], add=True)
            if with_count:
                pltpu.sync_copy(ones, cacc.at[gdst], add=True)
            refill_pad()

        for k in range(KPC):
            chunk_id = k * NC + c
            lo = chunk_id * CHUNK

            for i in range(CPT // LN):
                pltpu.sync_copy(zbuf, acc.at[pl.ds(s * CPT + i * LN, LN)])
            if with_count:
                for i in range(CPT // LN):
                    pltpu.sync_copy(zbufc, cacc.at[pl.ds(s * CPT + i * LN, LN)])
            @pl.when(s == 0)
            def _():
                pltpu.sync_copy(zbuf, acc.at[pl.ds(CHUNK, LN)])
                if with_count:
                    pltpu.sync_copy(zbufc, cacc.at[pl.ds(CHUNK, LN)])
            refill_pad()
            plsc.subcore_barrier()

            ebase = s * EPT

            def grp(gi, pend):
                off = gi * LN
                sv = src_v[pl.ds(off, LN)]
                dv = dst_v[pl.ds(off, LN)]
                m = (dv >= lo) & (dv < lo + CHUNK)
                mi = m.astype(jnp.int32)
                pos = pend + plsc.cumsum(mi) - 1
                plsc.store_scatter(gsrc, [pos], sv, mask=m)
                plsc.store_scatter(gdst, [pos], dv - lo, mask=m)
                pend = pend + jnp.sum(mi)
                full = pend > (G - LN)

                @pl.when(full)
                def _():
                    flush()
                return jnp.where(full, 0, pend)

            def blk(b, pend):
                pltpu.sync_copy(src_h.at[pl.ds(ebase + b * EB, EB)], src_v)
                pltpu.sync_copy(dst_h.at[pl.ds(ebase + b * EB, EB)], dst_v)
                return lax.fori_loop(0, GRP, grp, pend)

            lax.fori_loop(0, NB, blk, jnp.int32(0))
            flush()
            plsc.subcore_barrier()

            pltpu.sync_copy(acc.at[pl.ds(s * CPT, CPT)],
                            out_h.at[pl.ds(lo + s * CPT, CPT)])
            if with_count:
                pltpu.sync_copy(cacc.at[pl.ds(s * CPT, CPT)],
                                cnt_h.at[pl.ds(lo + s * CPT, CPT)])

    return pl.kernel(body, out_type=tuple(out_type), mesh=mesh,
                     scratch_types=scratch,
                     compiler_params=pltpu.CompilerParams(
                         needs_layout_passes=False,
                         use_tc_tiling_on_sc=False))


_CT = (((1,), (1,)), ((), ()))


def _dot(a, b):
    return lax.dot_general(a, b, _CT, preferred_element_type=jnp.float32)


@functools.cache
def _layer1_kernel(F_in):
    BR = 256

    def body(s_ref, c_ref, x_ref, a_ref, lw_ref, lb_ref, rw_ref,
             ww_ref, wb_ref, w2_ref, w2b_ref, in2_ref, r3_ref):
        cnt = jnp.maximum(c_ref[:, 0:1], 1.0)
        mean = s_ref[...] / cnt
        t = _dot(mean, lw_ref[...]) + lb_ref[...] + _dot(x_ref[...], rw_ref[...])
        a = a_ref[...]
        h1 = jnp.where(t >= 0, t, a * t)
        in2_ref[...] = h1 + _dot(x_ref[...], ww_ref[...]) + wb_ref[...]
        r3_ref[...] = h1 + _dot(x_ref[...], w2_ref[...]) + w2b_ref[...]

    grid = (NPAD // BR,)
    row = lambda i: (i, 0)
    fix = lambda i: (0, 0)
    return pl.pallas_call(
        body,
        grid=grid,
        in_specs=[
            pl.BlockSpec((BR, F_in), row),
            pl.BlockSpec((BR, LN), row),
            pl.BlockSpec((BR, F_in), row),
            pl.BlockSpec((1, 1), fix),
            pl.BlockSpec((H, F_in), fix),
            pl.BlockSpec((1, H), fix),
            pl.BlockSpec((H, F_in), fix),
            pl.BlockSpec((H, F_in), fix),
            pl.BlockSpec((1, H), fix),
            pl.BlockSpec((H, F_in), fix),
            pl.BlockSpec((1, H), fix),
        ],
        out_specs=[pl.BlockSpec((BR, H), row), pl.BlockSpec((BR, H), row)],
        out_shape=[jax.ShapeDtypeStruct((NPAD, H), jnp.float32)] * 2,
    )


@functools.cache
def _layer2_kernel():
    BR = 256

    def body(s_ref, c_ref, in_ref, r3_ref, a_ref, lw_ref, lb_ref, rw_ref,
             out_ref):
        cnt = jnp.maximum(c_ref[:, 0:1], 1.0)
        mean = s_ref[...].astype(jnp.float32) / cnt
        t = _dot(mean, lw_ref[...]) + lb_ref[...] + _dot(in_ref[...], rw_ref[...])
        a = a_ref[...]
        h2 = jnp.where(t >= 0, t, a * t)
        out_ref[...] = h2 + r3_ref[...]

    grid = (NPAD // 256,)
    row = lambda i: (i, 0)
    fix = lambda i: (0, 0)
    return pl.pallas_call(
        body,
        grid=grid,
        in_specs=[
            pl.BlockSpec((BR, H), row),
            pl.BlockSpec((BR, LN), row),
            pl.BlockSpec((BR, H), row),
            pl.BlockSpec((BR, H), row),
            pl.BlockSpec((1, 1), fix),
            pl.BlockSpec((H, H), fix),
            pl.BlockSpec((1, H), fix),
            pl.BlockSpec((H, H), fix),
        ],
        out_specs=pl.BlockSpec((BR, H), row),
        out_shape=jax.ShapeDtypeStruct((NPAD, H), jnp.float32),
    )


@functools.cache
def _layer3_kernel():
    BR = 256

    def body(s_ref, c_ref, in_ref, a_ref, lw_ref, lb_ref, rw_ref, out_ref):
        cnt = jnp.maximum(c_ref[:, 0:1], 1.0)
        mean = s_ref[...].astype(jnp.float32) / cnt
        t = _dot(mean, lw_ref[...]) + lb_ref[...] + _dot(in_ref[...], rw_ref[...])
        a = a_ref[...]
        out_ref[...] = jnp.where(t >= 0, t, a * t)

    grid = (NPAD // 256,)
    row = lambda i: (i, 0)
    fix = lambda i: (0, 0)
    return pl.pallas_call(
        body,
        grid=grid,
        in_specs=[
            pl.BlockSpec((BR, H), row),
            pl.BlockSpec((BR, LN), row),
            pl.BlockSpec((BR, H), row),
            pl.BlockSpec((1, 1), fix),
            pl.BlockSpec((H, H), fix),
            pl.BlockSpec((1, H), fix),
            pl.BlockSpec((H, H), fix),
        ],
        out_specs=pl.BlockSpec((BR, H), row),
        out_shape=jax.ShapeDtypeStruct((NPAD, H), jnp.float32),
    )


def _encode(x, ei, a2, lw1, lb1, rw1, lw2, lb2, rw2, lw3, lb3, rw3,
            Ww, Wb, W2w, W2b):
    src = ei[0]
    dst = ei[1]
    seg128 = _segsum_kernel(D, 2560, 4, True, N)
    seg512 = _segsum_kernel(H, 2560, 4, False, N, jnp.bfloat16)

    s1, c16 = seg128(src, dst, x)
    xp = jnp.pad(x, ((0, NPAD - N), (0, 0)))
    in2, r3 = _layer1_kernel(D)(s1, c16, xp, a2, lw1, lb1, rw1,
                                Ww, Wb, W2w, W2b)
    (s2,) = seg512(src, dst, in2.astype(jnp.bfloat16))
    in3 = _layer2_kernel()(s2, c16, in2, r3, a2, lw2, lb2, rw2)
    (s3,) = seg512(src, dst, in3.astype(jnp.bfloat16))
    h3 = _layer3_kernel()(s3, c16, in3, a2, lw3, lb3, rw3)
    return h3[:N]


def kernel(x1, x2, edge_index_v1, edge_index_v2, prelu_a, W_w, W_b, W2_w,
           W2_b, l1_lw, l1_lb, l1_rw, l2_lw, l2_lb, l2_rw, l3_lw, l3_lb,
           l3_rw):
    a2 = prelu_a.reshape(1, 1)
    lb1 = l1_lb.reshape(1, H)
    lb2 = l2_lb.reshape(1, H)
    lb3 = l3_lb.reshape(1, H)
    Wb = W_b.reshape(1, H)
    W2b = W2_b.reshape(1, H)
    args = (a2, l1_lw, lb1, l1_rw, l2_lw, lb2, l2_rw, l3_lw, lb3, l3_rw,
            W_w, Wb, W2_w, W2b)
    out1 = _encode(x1, edge_index_v1, *args)
    out1, x2b, ei2b = lax.optimization_barrier((out1, x2, edge_index_v2))
    out2 = _encode(x2b, ei2b, *args)
    return (out1, out2)


# speedup vs baseline: 3.8984x; 1.0482x over previous
import functools

import jax
import jax.numpy as jnp
from jax import lax
from jax.experimental import pallas as pl
from jax.experimental.pallas import tpu as pltpu
from jax.experimental.pallas import tpu_sc as plsc

NC = 2
NS = 16
LN = 16

N = 10000
NPAD = 10240
E = 320000
D = 128
H = 512

G = 128


@functools.cache
def _segsum_kernel(F, CHUNK, NCHUNK, with_count, n_rows, dtype=jnp.float32):
    CPT = CHUNK // NS
    KPC = NCHUNK // NC
    EPT = E // NS
    EB = 2000
    NB = EPT // EB
    GRP = EB // LN
    FV = F // LN

    assert CHUNK % NS == 0 and NCHUNK % NC == 0 and NCHUNK * CHUNK == NPAD
    assert EPT % EB == 0 and EB % LN == 0 and CPT % LN == 0

    mesh = plsc.VectorSubcoreMesh(core_axis_name="c", subcore_axis_name="s")

    out_type = [jax.ShapeDtypeStruct((NPAD, F), dtype)]
    if with_count:
        out_type.append(jax.ShapeDtypeStruct((NPAD, LN), jnp.float32))

    scratch = [
        pltpu.VMEM((EB,), jnp.int32),
        pltpu.VMEM((EB,), jnp.int32),
        pltpu.VMEM((G,), jnp.int32),
        pltpu.VMEM((G,), jnp.int32),
        pltpu.VMEM((G, F), dtype),
        pltpu.VMEM((LN, F), dtype),
        pltpu.VMEM_SHARED((CHUNK + LN, F), dtype),
        pltpu.SemaphoreType.DMA,
    ]
    if with_count:
        scratch += [
            pltpu.VMEM((G, LN), jnp.float32),
            pltpu.VMEM((LN, LN), jnp.float32),
            pltpu.VMEM_SHARED((CHUNK + LN, LN), jnp.float32),
        ]

    def body(src_h, dst_h, x_h, out_h, *rest):
        if with_count:
            (cnt_h, src_v, dst_v, gsrc, gdst, rows, zbuf, acc, sem,
             ones, zbufc, cacc) = rest
        else:
            (src_v, dst_v, gsrc, gdst, rows, zbuf, acc, sem) = rest

        c = lax.axis_index("c")
        s = lax.axis_index("s")
        zero16 = jnp.zeros((LN,), jnp.float32)
        izero = jnp.zeros((LN,), jnp.int32)
        pad_src = izero + s
        pad_dst = izero + (CHUNK + s)

        ZW = LN if dtype == jnp.float32 else 2 * LN
        zvec = jnp.zeros((ZW,), dtype)

        def zb(k, _):
            zbuf[k // (F // ZW), pl.ds((k % (F // ZW)) * ZW, ZW)] = zvec
            return 0
        lax.fori_loop(0, LN * (F // ZW), zb, 0)
        if with_count:
            one16 = zero16 + 1.0

            def ob(k, _):
                ones[k, :] = one16
                return 0
            lax.fori_loop(0, G, ob, 0)

            def zc(k, _):
                zbufc[k, :] = zero16
                return 0
            lax.fori_loop(0, LN, zc, 0)

        def refill_pad():
            for j in range(G // LN):
                gsrc[pl.ds(j * LN, LN)] = pad_src
                gdst[pl.ds(j * LN, LN)] = pad_dst

        def flush():
            pltpu.async_copy(x_h.at[gsrc], rows, sem).wait()
            pltpu.sync_copy(rows, acc.at[gdst], add=True)
            if with_count:
                pltpu.sync_copy(ones, cacc.at[gdst], add=True)
            refill_pad()

        for k in range(KPC):
            chunk_id = k * NC + c
            lo = chunk_id * CHUNK

            for i in range(CPT // LN):
                pltpu.sync_copy(zbuf, acc.at[pl.ds(s * CPT + i * LN, LN)])
            if with_count:
                for i in range(CPT // LN):
                    pltpu.sync_copy(zbufc, cacc.at[pl.ds(s * CPT + i * LN, LN)])
            @pl.when(s == 0)
            def _():
                pltpu.sync_copy(zbuf, acc.at[pl.ds(CHUNK, LN)])
                if with_count:
                    pltpu.sync_copy(zbufc, cacc.at[pl.ds(CHUNK, LN)])
            refill_pad()
            plsc.subcore_barrier()

            ebase = s * EPT

            def grp(gi, pend):
                off = gi * LN
                sv = src_v[pl.ds(off, LN)]
                dv = dst_v[pl.ds(off, LN)]
                m = (dv >= lo) & (dv < lo + CHUNK)
                mi = m.astype(jnp.int32)
                pos = pend + plsc.cumsum(mi) - 1
                plsc.store_scatter(gsrc, [pos], sv, mask=m)
                plsc.store_scatter(gdst, [pos], dv - lo, mask=m)
                pend = pend + jnp.sum(mi)
                full = pend > (G - LN)

                @pl.when(full)
                def _():
                    flush()
                return jnp.where(full, 0, pend)

            def blk(b, pend):
                pltpu.sync_copy(src_h.at[pl.ds(ebase + b * EB, EB)], src_v)
                pltpu.sync_copy(dst_h.at[pl.ds(ebase + b * EB, EB)], dst_v)
                return lax.fori_loop(0, GRP, grp, pend)

            lax.fori_loop(0, NB, blk, jnp.int32(0))
            flush()
            plsc.subcore_barrier()

            pltpu.sync_copy(acc.at[pl.ds(s * CPT, CPT)],
                            out_h.at[pl.ds(lo + s * CPT, CPT)])
            if with_count:
                pltpu.sync_copy(cacc.at[pl.ds(s * CPT, CPT)],
                                cnt_h.at[pl.ds(lo + s * CPT, CPT)])

    return pl.kernel(body, out_type=tuple(out_type), mesh=mesh,
                     scratch_types=scratch,
                     compiler_params=pltpu.CompilerParams(
                         needs_layout_passes=False,
                         use_tc_tiling_on_sc=False))


_CT = (((1,), (1,)), ((), ()))


def _dot(a, b):
    return lax.dot_general(a, b, _CT, preferred_element_type=jnp.float32)


@functools.cache
def _layer1_kernel(F_in):
    BR = 256

    def body(s_ref, c_ref, x_ref, a_ref, lw_ref, lb_ref, rw_ref,
             ww_ref, wb_ref, w2_ref, w2b_ref, in2_ref, r3_ref):
        cnt = jnp.maximum(c_ref[:, 0:1], 1.0)
        mean = s_ref[...] / cnt
        t = _dot(mean, lw_ref[...]) + lb_ref[...] + _dot(x_ref[...], rw_ref[...])
        a = a_ref[...]
        h1 = jnp.where(t >= 0, t, a * t)
        in2_ref[...] = h1 + _dot(x_ref[...], ww_ref[...]) + wb_ref[...]
        r3_ref[...] = h1 + _dot(x_ref[...], w2_ref[...]) + w2b_ref[...]

    grid = (NPAD // BR,)
    row = lambda i: (i, 0)
    fix = lambda i: (0, 0)
    return pl.pallas_call(
        body,
        grid=grid,
        in_specs=[
            pl.BlockSpec((BR, F_in), row),
            pl.BlockSpec((BR, LN), row),
            pl.BlockSpec((BR, F_in), row),
            pl.BlockSpec((1, 1), fix),
            pl.BlockSpec((H, F_in), fix),
            pl.BlockSpec((1, H), fix),
            pl.BlockSpec((H, F_in), fix),
            pl.BlockSpec((H, F_in), fix),
            pl.BlockSpec((1, H), fix),
            pl.BlockSpec((H, F_in), fix),
            pl.BlockSpec((1, H), fix),
        ],
        out_specs=[pl.BlockSpec((BR, H), row), pl.BlockSpec((BR, H), row)],
        out_shape=[jax.ShapeDtypeStruct((NPAD, H), jnp.float32)] * 2,
    )


@functools.cache
def _layer2_kernel():
    BR = 256

    def body(s_ref, c_ref, in_ref, r3_ref, a_ref, lw_ref, lb_ref, rw_ref,
             out_ref):
        cnt = jnp.maximum(c_ref[:, 0:1], 1.0)
        mean = s_ref[...].astype(jnp.float32) / cnt
        t = _dot(mean, lw_ref[...]) + lb_ref[...] + _dot(in_ref[...], rw_ref[...])
        a = a_ref[...]
        h2 = jnp.where(t >= 0, t, a * t)
        out_ref[...] = h2 + r3_ref[...]

    grid = (NPAD // 256,)
    row = lambda i: (i, 0)
    fix = lambda i: (0, 0)
    return pl.pallas_call(
        body,
        grid=grid,
        in_specs=[
            pl.BlockSpec((BR, H), row),
            pl.BlockSpec((BR, LN), row),
            pl.BlockSpec((BR, H), row),
            pl.BlockSpec((BR, H), row),
            pl.BlockSpec((1, 1), fix),
            pl.BlockSpec((H, H), fix),
            pl.BlockSpec((1, H), fix),
            pl.BlockSpec((H, H), fix),
        ],
        out_specs=pl.BlockSpec((BR, H), row),
        out_shape=jax.ShapeDtypeStruct((NPAD, H), jnp.float32),
    )


@functools.cache
def _layer3_kernel():
    BR = 256

    def body(s_ref, c_ref, in_ref, a_ref, lw_ref, lb_ref, rw_ref, out_ref):
        cnt = jnp.maximum(c_ref[:, 0:1], 1.0)
        mean = s_ref[...].astype(jnp.float32) / cnt
        t = _dot(mean, lw_ref[...]) + lb_ref[...] + _dot(in_ref[...], rw_ref[...])
        a = a_ref[...]
        out_ref[...] = jnp.where(t >= 0, t, a * t)

    grid = (NPAD // 256,)
    row = lambda i: (i, 0)
    fix = lambda i: (0, 0)
    return pl.pallas_call(
        body,
        grid=grid,
        in_specs=[
            pl.BlockSpec((BR, H), row),
            pl.BlockSpec((BR, LN), row),
            pl.BlockSpec((BR, H), row),
            pl.BlockSpec((1, 1), fix),
            pl.BlockSpec((H, H), fix),
            pl.BlockSpec((1, H), fix),
            pl.BlockSpec((H, H), fix),
        ],
        out_specs=pl.BlockSpec((BR, H), row),
        out_shape=jax.ShapeDtypeStruct((NPAD, H), jnp.float32),
    )


def _encode(x, ei, a2, lw1, lb1, rw1, lw2, lb2, rw2, lw3, lb3, rw3,
            Ww, Wb, W2w, W2b):
    src = ei[0]
    dst = ei[1]
    seg128 = _segsum_kernel(D, 5120, 2, True, N)
    seg512 = _segsum_kernel(H, 2560, 4, False, N, jnp.bfloat16)

    s1, c16 = seg128(src, dst, x)
    xp = jnp.pad(x, ((0, NPAD - N), (0, 0)))
    in2, r3 = _layer1_kernel(D)(s1, c16, xp, a2, lw1, lb1, rw1,
                                Ww, Wb, W2w, W2b)
    (s2,) = seg512(src, dst, in2.astype(jnp.bfloat16))
    in3 = _layer2_kernel()(s2, c16, in2, r3, a2, lw2, lb2, rw2)
    (s3,) = seg512(src, dst, in3.astype(jnp.bfloat16))
    h3 = _layer3_kernel()(s3, c16, in3, a2, lw3, lb3, rw3)
    return h3[:N]


def kernel(x1, x2, edge_index_v1, edge_index_v2, prelu_a, W_w, W_b, W2_w,
           W2_b, l1_lw, l1_lb, l1_rw, l2_lw, l2_lb, l2_rw, l3_lw, l3_lb,
           l3_rw):
    a2 = prelu_a.reshape(1, 1)
    lb1 = l1_lb.reshape(1, H)
    lb2 = l2_lb.reshape(1, H)
    lb3 = l3_lb.reshape(1, H)
    Wb = W_b.reshape(1, H)
    W2b = W2_b.reshape(1, H)
    args = (a2, l1_lw, lb1, l1_rw, l2_lw, lb2, l2_rw, l3_lw, lb3, l3_rw,
            W_w, Wb, W2_w, W2b)
    out1 = _encode(x1, edge_index_v1, *args)
    out1, x2b, ei2b = lax.optimization_barrier((out1, x2, edge_index_v2))
    out2 = _encode(x2b, ei2b, *args)
    return (out1, out2)


# speedup vs baseline: 4.1662x; 1.0687x over previous
import functools

import jax
import jax.numpy as jnp
from jax import lax
from jax.experimental import pallas as pl
from jax.experimental.pallas import tpu as pltpu
from jax.experimental.pallas import tpu_sc as plsc

NC = 2
NS = 16
LN = 16

N = 10000
NPAD = 10240
E = 320000
D = 128
H = 512

G = 256


@functools.cache
def _segsum_kernel(F, CHUNK, NCHUNK, with_count, n_rows, dtype=jnp.float32):
    CPT = CHUNK // NS
    KPC = NCHUNK // NC
    EPT = E // NS
    EB = 2000
    NB = EPT // EB
    GRP = EB // LN
    FV = F // LN

    assert CHUNK % NS == 0 and NCHUNK % NC == 0 and NCHUNK * CHUNK == NPAD
    assert EPT % EB == 0 and EB % LN == 0 and CPT % LN == 0

    mesh = plsc.VectorSubcoreMesh(core_axis_name="c", subcore_axis_name="s")

    out_type = [jax.ShapeDtypeStruct((NPAD, F), dtype)]
    if with_count:
        out_type.append(jax.ShapeDtypeStruct((NPAD, LN), jnp.float32))

    scratch = [
        pltpu.VMEM((EB,), jnp.int32),
        pltpu.VMEM((EB,), jnp.int32),
        pltpu.VMEM((G,), jnp.int32),
        pltpu.VMEM((G,), jnp.int32),
        pltpu.VMEM((G, F), dtype),
        pltpu.VMEM((LN, F), dtype),
        pltpu.VMEM_SHARED((CHUNK + LN, F), dtype),
        pltpu.SemaphoreType.DMA,
    ]
    if with_count:
        scratch += [
            pltpu.VMEM((G, LN), jnp.float32),
            pltpu.VMEM((LN, LN), jnp.float32),
            pltpu.VMEM_SHARED((CHUNK + LN, LN), jnp.float32),
        ]

    def body(src_h, dst_h, x_h, out_h, *rest):
        if with_count:
            (cnt_h, src_v, dst_v, gsrc, gdst, rows, zbuf, acc, sem,
             ones, zbufc, cacc) = rest
        else:
            (src_v, dst_v, gsrc, gdst, rows, zbuf, acc, sem) = rest

        c = lax.axis_index("c")
        s = lax.axis_index("s")
        zero16 = jnp.zeros((LN,), jnp.float32)
        izero = jnp.zeros((LN,), jnp.int32)
        pad_src = izero + s
        pad_dst = izero + (CHUNK + s)

        ZW = LN if dtype == jnp.float32 else 2 * LN
        zvec = jnp.zeros((ZW,), dtype)

        def zb(k, _):
            zbuf[k // (F // ZW), pl.ds((k % (F // ZW)) * ZW, ZW)] = zvec
            return 0
        lax.fori_loop(0, LN * (F // ZW), zb, 0)
        if with_count:
            one16 = zero16 + 1.0

            def ob(k, _):
                ones[k, :] = one16
                return 0
            lax.fori_loop(0, G, ob, 0)

            def zc(k, _):
                zbufc[k, :] = zero16
                return 0
            lax.fori_loop(0, LN, zc, 0)

        def refill_pad():
            for j in range(G // LN):
                gsrc[pl.ds(j * LN, LN)] = pad_src
                gdst[pl.ds(j * LN, LN)] = pad_dst

        def flush():
            pltpu.async_copy(x_h.at[gsrc], rows, sem).wait()
            pltpu.sync_copy(rows, acc.at[gdst], add=True)
            if with_count:
                pltpu.sync_copy(ones, cacc.at[gdst], add=True)
            refill_pad()

        for k in range(KPC):
            chunk_id = k * NC + c
            lo = chunk_id * CHUNK

            for i in range(CPT // LN):
                pltpu.sync_copy(zbuf, acc.at[pl.ds(s * CPT + i * LN, LN)])
            if with_count:
                for i in range(CPT // LN):
                    pltpu.sync_copy(zbufc, cacc.at[pl.ds(s * CPT + i * LN, LN)])
            @pl.when(s == 0)
            def _():
                pltpu.sync_copy(zbuf, acc.at[pl.ds(CHUNK, LN)])
                if with_count:
                    pltpu.sync_copy(zbufc, cacc.at[pl.ds(CHUNK, LN)])
            refill_pad()
            plsc.subcore_barrier()

            ebase = s * EPT

            def grp(gi, pend):
                off = gi * LN
                sv = src_v[pl.ds(off, LN)]
                dv = dst_v[pl.ds(off, LN)]
                m = (dv >= lo) & (dv < lo + CHUNK)
                mi = m.astype(jnp.int32)
                pos = pend + plsc.cumsum(mi) - 1
                plsc.store_scatter(gsrc, [pos], sv, mask=m)
                plsc.store_scatter(gdst, [pos], dv - lo, mask=m)
                pend = pend + jnp.sum(mi)
                full = pend > (G - LN)

                @pl.when(full)
                def _():
                    flush()
                return jnp.where(full, 0, pend)

            def blk(b, pend):
                pltpu.sync_copy(src_h.at[pl.ds(ebase + b * EB, EB)], src_v)
                pltpu.sync_copy(dst_h.at[pl.ds(ebase + b * EB, EB)], dst_v)
                return lax.fori_loop(0, GRP, grp, pend)

            lax.fori_loop(0, NB, blk, jnp.int32(0))
            flush()
            plsc.subcore_barrier()

            pltpu.sync_copy(acc.at[pl.ds(s * CPT, CPT)],
                            out_h.at[pl.ds(lo + s * CPT, CPT)])
            if with_count:
                pltpu.sync_copy(cacc.at[pl.ds(s * CPT, CPT)],
                                cnt_h.at[pl.ds(lo + s * CPT, CPT)])

    return pl.kernel(body, out_type=tuple(out_type), mesh=mesh,
                     scratch_types=scratch,
                     compiler_params=pltpu.CompilerParams(
                         needs_layout_passes=False,
                         use_tc_tiling_on_sc=False))


_CT = (((1,), (1,)), ((), ()))


def _dot(a, b):
    return lax.dot_general(a, b, _CT, preferred_element_type=jnp.float32)


@functools.cache
def _layer1_kernel(F_in):
    BR = 256

    def body(s_ref, c_ref, x_ref, a_ref, lw_ref, lb_ref, rw_ref,
             ww_ref, wb_ref, w2_ref, w2b_ref, in2_ref, r3_ref):
        cnt = jnp.maximum(c_ref[:, 0:1], 1.0)
        mean = s_ref[...] / cnt
        t = _dot(mean, lw_ref[...]) + lb_ref[...] + _dot(x_ref[...], rw_ref[...])
        a = a_ref[...]
        h1 = jnp.where(t >= 0, t, a * t)
        in2_ref[...] = h1 + _dot(x_ref[...], ww_ref[...]) + wb_ref[...]
        r3_ref[...] = h1 + _dot(x_ref[...], w2_ref[...]) + w2b_ref[...]

    grid = (NPAD // BR,)
    row = lambda i: (i, 0)
    fix = lambda i: (0, 0)
    return pl.pallas_call(
        body,
        grid=grid,
        in_specs=[
            pl.BlockSpec((BR, F_in), row),
            pl.BlockSpec((BR, LN), row),
            pl.BlockSpec((BR, F_in), row),
            pl.BlockSpec((1, 1), fix),
            pl.BlockSpec((H, F_in), fix),
            pl.BlockSpec((1, H), fix),
            pl.BlockSpec((H, F_in), fix),
            pl.BlockSpec((H, F_in), fix),
            pl.BlockSpec((1, H), fix),
            pl.BlockSpec((H, F_in), fix),
            pl.BlockSpec((1, H), fix),
        ],
        out_specs=[pl.BlockSpec((BR, H), row), pl.BlockSpec((BR, H), row)],
        out_shape=[jax.ShapeDtypeStruct((NPAD, H), jnp.float32)] * 2,
    )


@functools.cache
def _layer2_kernel():
    BR = 256

    def body(s_ref, c_ref, in_ref, r3_ref, a_ref, lw_ref, lb_ref, rw_ref,
             out_ref):
        cnt = jnp.maximum(c_ref[:, 0:1], 1.0)
        mean = s_ref[...].astype(jnp.float32) / cnt
        t = _dot(mean, lw_ref[...]) + lb_ref[...] + _dot(in_ref[...], rw_ref[...])
        a = a_ref[...]
        h2 = jnp.where(t >= 0, t, a * t)
        out_ref[...] = h2 + r3_ref[...]

    grid = (NPAD // 256,)
    row = lambda i: (i, 0)
    fix = lambda i: (0, 0)
    return pl.pallas_call(
        body,
        grid=grid,
        in_specs=[
            pl.BlockSpec((BR, H), row),
            pl.BlockSpec((BR, LN), row),
            pl.BlockSpec((BR, H), row),
            pl.BlockSpec((BR, H), row),
            pl.BlockSpec((1, 1), fix),
            pl.BlockSpec((H, H), fix),
            pl.BlockSpec((1, H), fix),
            pl.BlockSpec((H, H), fix),
        ],
        out_specs=pl.BlockSpec((BR, H), row),
        out_shape=jax.ShapeDtypeStruct((NPAD, H), jnp.float32),
    )


@functools.cache
def _layer3_kernel():
    BR = 256

    def body(s_ref, c_ref, in_ref, a_ref, lw_ref, lb_ref, rw_ref, out_ref):
        cnt = jnp.maximum(c_ref[:, 0:1], 1.0)
        mean = s_ref[...].astype(jnp.float32) / cnt
        t = _dot(mean, lw_ref[...]) + lb_ref[...] + _dot(in_ref[...], rw_ref[...])
        a = a_ref[...]
        out_ref[...] = jnp.where(t >= 0, t, a * t)

    grid = (NPAD // 256,)
    row = lambda i: (i, 0)
    fix = lambda i: (0, 0)
    return pl.pallas_call(
        body,
        grid=grid,
        in_specs=[
            pl.BlockSpec((BR, H), row),
            pl.BlockSpec((BR, LN), row),
            pl.BlockSpec((BR, H), row),
            pl.BlockSpec((1, 1), fix),
            pl.BlockSpec((H, H), fix),
            pl.BlockSpec((1, H), fix),
            pl.BlockSpec((H, H), fix),
        ],
        out_specs=pl.BlockSpec((BR, H), row),
        out_shape=jax.ShapeDtypeStruct((NPAD, H), jnp.float32),
    )


def _encode(x, ei, a2, lw1, lb1, rw1, lw2, lb2, rw2, lw3, lb3, rw3,
            Ww, Wb, W2w, W2b):
    src = ei[0]
    dst = ei[1]
    seg128 = _segsum_kernel(D, 5120, 2, True, N)
    seg512 = _segsum_kernel(H, 2560, 4, False, N, jnp.bfloat16)

    s1, c16 = seg128(src, dst, x)
    xp = jnp.pad(x, ((0, NPAD - N), (0, 0)))
    in2, r3 = _layer1_kernel(D)(s1, c16, xp, a2, lw1, lb1, rw1,
                                Ww, Wb, W2w, W2b)
    (s2,) = seg512(src, dst, in2.astype(jnp.bfloat16))
    in3 = _layer2_kernel()(s2, c16, in2, r3, a2, lw2, lb2, rw2)
    (s3,) = seg512(src, dst, in3.astype(jnp.bfloat16))
    h3 = _layer3_kernel()(s3, c16, in3, a2, lw3, lb3, rw3)
    return h3[:N]


def kernel(x1, x2, edge_index_v1, edge_index_v2, prelu_a, W_w, W_b, W2_w,
           W2_b, l1_lw, l1_lb, l1_rw, l2_lw, l2_lb, l2_rw, l3_lw, l3_lb,
           l3_rw):
    a2 = prelu_a.reshape(1, 1)
    lb1 = l1_lb.reshape(1, H)
    lb2 = l2_lb.reshape(1, H)
    lb3 = l3_lb.reshape(1, H)
    Wb = W_b.reshape(1, H)
    W2b = W2_b.reshape(1, H)
    args = (a2, l1_lw, lb1, l1_rw, l2_lw, lb2, l2_rw, l3_lw, lb3, l3_rw,
            W_w, Wb, W2_w, W2b)
    out1 = _encode(x1, edge_index_v1, *args)
    out1, x2b, ei2b = lax.optimization_barrier((out1, x2, edge_index_v2))
    out2 = _encode(x2b, ei2b, *args)
    return (out1, out2)


# speedup vs baseline: 4.2227x; 1.0136x over previous
import functools

import jax
import jax.numpy as jnp
from jax import lax
from jax.experimental import pallas as pl
from jax.experimental.pallas import tpu as pltpu
from jax.experimental.pallas import tpu_sc as plsc

NC = 2
NS = 16
LN = 16

N = 10000
NPAD = 10240
E = 320000
D = 128
H = 512

G = 256


@functools.cache
def _segsum_kernel(F, CHUNK, NCHUNK, with_count, n_rows, dtype=jnp.float32):
    CPT = CHUNK // NS
    KPC = NCHUNK // NC
    EPT = E // NS
    EB = 2000
    NB = EPT // EB
    GRP = EB // LN
    FV = F // LN

    assert CHUNK % NS == 0 and NCHUNK % NC == 0 and NCHUNK * CHUNK == NPAD
    assert EPT % EB == 0 and EB % LN == 0 and CPT % LN == 0

    mesh = plsc.VectorSubcoreMesh(core_axis_name="c", subcore_axis_name="s")

    out_type = [jax.ShapeDtypeStruct((NPAD, F), dtype)]
    if with_count:
        out_type.append(jax.ShapeDtypeStruct((NPAD, LN), jnp.float32))

    scratch = [
        pltpu.VMEM((EB,), jnp.int32),
        pltpu.VMEM((EB,), jnp.int32),
        pltpu.VMEM((G // 2,), jnp.int32),
        pltpu.VMEM((G // 2,), jnp.int32),
        pltpu.VMEM((G // 2,), jnp.int32),
        pltpu.VMEM((G // 2,), jnp.int32),
        pltpu.VMEM((G // 2, F), dtype),
        pltpu.VMEM((G // 2, F), dtype),
        pltpu.VMEM((LN, F), dtype),
        pltpu.VMEM_SHARED((CHUNK + LN, F), dtype),
        pltpu.SemaphoreType.DMA,
        pltpu.SemaphoreType.DMA,
    ]
    if with_count:
        scratch += [
            pltpu.VMEM((G // 2, LN), jnp.float32),
            pltpu.VMEM((LN, LN), jnp.float32),
            pltpu.VMEM_SHARED((CHUNK + LN, LN), jnp.float32),
        ]

    def body(src_h, dst_h, x_h, out_h, *rest):
        if with_count:
            (cnt_h, src_v, dst_v, gsrcA, gsrcB, gdstA, gdstB, rowsA, rowsB,
             zbuf, acc, sem, sem2, ones, zbufc, cacc) = rest
        else:
            (src_v, dst_v, gsrcA, gsrcB, gdstA, gdstB, rowsA, rowsB,
             zbuf, acc, sem, sem2) = rest
        GH = G // 2

        c = lax.axis_index("c")
        s = lax.axis_index("s")
        zero16 = jnp.zeros((LN,), jnp.float32)
        izero = jnp.zeros((LN,), jnp.int32)
        pad_src = izero + s
        pad_dst = izero + (CHUNK + s)

        ZW = LN if dtype == jnp.float32 else 2 * LN
        zvec = jnp.zeros((ZW,), dtype)

        def zb(k, _):
            zbuf[k // (F // ZW), pl.ds((k % (F // ZW)) * ZW, ZW)] = zvec
            return 0
        lax.fori_loop(0, LN * (F // ZW), zb, 0)
        if with_count:
            one16 = zero16 + 1.0

            def ob(k, _):
                ones[k, :] = one16
                return 0
            lax.fori_loop(0, G // 2, ob, 0)

            def zc(k, _):
                zbufc[k, :] = zero16
                return 0
            lax.fori_loop(0, LN, zc, 0)

        def refill_pad():
            for j in range(GH // LN):
                gsrcA[pl.ds(j * LN, LN)] = pad_src
                gsrcB[pl.ds(j * LN, LN)] = pad_src
                gdstA[pl.ds(j * LN, LN)] = pad_dst
                gdstB[pl.ds(j * LN, LN)] = pad_dst

        def flush():
            pltpu.async_copy(x_h.at[gsrcA], rowsA, sem).wait()
            sa = pltpu.async_copy(rowsA, acc.at[gdstA], sem2, add=True)
            gb = pltpu.async_copy(x_h.at[gsrcB], rowsB, sem)
            sa.wait()
            gb.wait()
            pltpu.sync_copy(rowsB, acc.at[gdstB], add=True)
            if with_count:
                pltpu.sync_copy(ones, cacc.at[gdstA], add=True)
                pltpu.sync_copy(ones, cacc.at[gdstB], add=True)
            refill_pad()

        for k in range(KPC):
            chunk_id = k * NC + c
            lo = chunk_id * CHUNK

            for i in range(CPT // LN):
                pltpu.sync_copy(zbuf, acc.at[pl.ds(s * CPT + i * LN, LN)])
            if with_count:
                for i in range(CPT // LN):
                    pltpu.sync_copy(zbufc, cacc.at[pl.ds(s * CPT + i * LN, LN)])
            @pl.when(s == 0)
            def _():
                pltpu.sync_copy(zbuf, acc.at[pl.ds(CHUNK, LN)])
                if with_count:
                    pltpu.sync_copy(zbufc, cacc.at[pl.ds(CHUNK, LN)])
            refill_pad()
            plsc.subcore_barrier()

            ebase = s * EPT

            def grp(gi, pend):
                off = gi * LN
                sv = src_v[pl.ds(off, LN)]
                dv = dst_v[pl.ds(off, LN)]
                m = (dv >= lo) & (dv < lo + CHUNK)
                mi = m.astype(jnp.int32)
                pos = pend + plsc.cumsum(mi) - 1
                doff = dv - lo
                mA = m & (pos < GH)
                mB = m & (pos >= GH)
                posB = pos - GH
                plsc.store_scatter(gsrcA, [pos], sv, mask=mA)
                plsc.store_scatter(gdstA, [pos], doff, mask=mA)
                plsc.store_scatter(gsrcB, [posB], sv, mask=mB)
                plsc.store_scatter(gdstB, [posB], doff, mask=mB)
                pend = pend + jnp.sum(mi)
                full = pend > (G - LN)

                @pl.when(full)
                def _():
                    flush()
                return jnp.where(full, 0, pend)

            def blk(b, pend):
                pltpu.sync_copy(src_h.at[pl.ds(ebase + b * EB, EB)], src_v)
                pltpu.sync_copy(dst_h.at[pl.ds(ebase + b * EB, EB)], dst_v)
                return lax.fori_loop(0, GRP, grp, pend)

            lax.fori_loop(0, NB, blk, jnp.int32(0))
            flush()
            plsc.subcore_barrier()

            pltpu.sync_copy(acc.at[pl.ds(s * CPT, CPT)],
                            out_h.at[pl.ds(lo + s * CPT, CPT)])
            if with_count:
                pltpu.sync_copy(cacc.at[pl.ds(s * CPT, CPT)],
                                cnt_h.at[pl.ds(lo + s * CPT, CPT)])

    return pl.kernel(body, out_type=tuple(out_type), mesh=mesh,
                     scratch_types=scratch,
                     compiler_params=pltpu.CompilerParams(
                         needs_layout_passes=False,
                         use_tc_tiling_on_sc=False))


_CT = (((1,), (1,)), ((), ()))


def _dot(a, b):
    return lax.dot_general(a, b, _CT, preferred_element_type=jnp.float32)


@functools.cache
def _layer1_kernel(F_in):
    BR = 256

    def body(s_ref, c_ref, x_ref, a_ref, lw_ref, lb_ref, rw_ref,
             ww_ref, wb_ref, w2_ref, w2b_ref, in2_ref, r3_ref):
        cnt = jnp.maximum(c_ref[:, 0:1], 1.0)
        mean = s_ref[...] / cnt
        t = _dot(mean, lw_ref[...]) + lb_ref[...] + _dot(x_ref[...], rw_ref[...])
        a = a_ref[...]
        h1 = jnp.where(t >= 0, t, a * t)
        in2_ref[...] = h1 + _dot(x_ref[...], ww_ref[...]) + wb_ref[...]
        r3_ref[...] = h1 + _dot(x_ref[...], w2_ref[...]) + w2b_ref[...]

    grid = (NPAD // BR,)
    row = lambda i: (i, 0)
    fix = lambda i: (0, 0)
    return pl.pallas_call(
        body,
        grid=grid,
        in_specs=[
            pl.BlockSpec((BR, F_in), row),
            pl.BlockSpec((BR, LN), row),
            pl.BlockSpec((BR, F_in), row),
            pl.BlockSpec((1, 1), fix),
            pl.BlockSpec((H, F_in), fix),
            pl.BlockSpec((1, H), fix),
            pl.BlockSpec((H, F_in), fix),
            pl.BlockSpec((H, F_in), fix),
            pl.BlockSpec((1, H), fix),
            pl.BlockSpec((H, F_in), fix),
            pl.BlockSpec((1, H), fix),
        ],
        out_specs=[pl.BlockSpec((BR, H), row), pl.BlockSpec((BR, H), row)],
        out_shape=[jax.ShapeDtypeStruct((NPAD, H), jnp.float32)] * 2,
    )


@functools.cache
def _layer2_kernel():
    BR = 256

    def body(s_ref, c_ref, in_ref, r3_ref, a_ref, lw_ref, lb_ref, rw_ref,
             out_ref):
        cnt = jnp.maximum(c_ref[:, 0:1], 1.0)
        mean = s_ref[...].astype(jnp.float32) / cnt
        t = _dot(mean, lw_ref[...]) + lb_ref[...] + _dot(in_ref[...], rw_ref[...])
        a = a_ref[...]
        h2 = jnp.where(t >= 0, t, a * t)
        out_ref[...] = h2 + r3_ref[...]

    grid = (NPAD // 256,)
    row = lambda i: (i, 0)
    fix = lambda i: (0, 0)
    return pl.pallas_call(
        body,
        grid=grid,
        in_specs=[
            pl.BlockSpec((BR, H), row),
            pl.BlockSpec((BR, LN), row),
            pl.BlockSpec((BR, H), row),
            pl.BlockSpec((BR, H), row),
            pl.BlockSpec((1, 1), fix),
            pl.BlockSpec((H, H), fix),
            pl.BlockSpec((1, H), fix),
            pl.BlockSpec((H, H), fix),
        ],
        out_specs=pl.BlockSpec((BR, H), row),
        out_shape=jax.ShapeDtypeStruct((NPAD, H), jnp.float32),
    )


@functools.cache
def _layer3_kernel():
    BR = 256

    def body(s_ref, c_ref, in_ref, a_ref, lw_ref, lb_ref, rw_ref, out_ref):
        cnt = jnp.maximum(c_ref[:, 0:1], 1.0)
        mean = s_ref[...].astype(jnp.float32) / cnt
        t = _dot(mean, lw_ref[...]) + lb_ref[...] + _dot(in_ref[...], rw_ref[...])
        a = a_ref[...]
        out_ref[...] = jnp.where(t >= 0, t, a * t)

    grid = (NPAD // 256,)
    row = lambda i: (i, 0)
    fix = lambda i: (0, 0)
    return pl.pallas_call(
        body,
        grid=grid,
        in_specs=[
            pl.BlockSpec((BR, H), row),
            pl.BlockSpec((BR, LN), row),
            pl.BlockSpec((BR, H), row),
            pl.BlockSpec((1, 1), fix),
            pl.BlockSpec((H, H), fix),
            pl.BlockSpec((1, H), fix),
            pl.BlockSpec((H, H), fix),
        ],
        out_specs=pl.BlockSpec((BR, H), row),
        out_shape=jax.ShapeDtypeStruct((NPAD, H), jnp.float32),
    )


def _encode(x, ei, a2, lw1, lb1, rw1, lw2, lb2, rw2, lw3, lb3, rw3,
            Ww, Wb, W2w, W2b):
    src = ei[0]
    dst = ei[1]
    seg128 = _segsum_kernel(D, 5120, 2, True, N)
    seg512 = _segsum_kernel(H, 2560, 4, False, N, jnp.bfloat16)

    s1, c16 = seg128(src, dst, x)
    xp = jnp.pad(x, ((0, NPAD - N), (0, 0)))
    in2, r3 = _layer1_kernel(D)(s1, c16, xp, a2, lw1, lb1, rw1,
                                Ww, Wb, W2w, W2b)
    (s2,) = seg512(src, dst, in2.astype(jnp.bfloat16))
    in3 = _layer2_kernel()(s2, c16, in2, r3, a2, lw2, lb2, rw2)
    (s3,) = seg512(src, dst, in3.astype(jnp.bfloat16))
    h3 = _layer3_kernel()(s3, c16, in3, a2, lw3, lb3, rw3)
    return h3[:N]


def kernel(x1, x2, edge_index_v1, edge_index_v2, prelu_a, W_w, W_b, W2_w,
           W2_b, l1_lw, l1_lb, l1_rw, l2_lw, l2_lb, l2_rw, l3_lw, l3_lb,
           l3_rw):
    a2 = prelu_a.reshape(1, 1)
    lb1 = l1_lb.reshape(1, H)
    lb2 = l2_lb.reshape(1, H)
    lb3 = l3_lb.reshape(1, H)
    Wb = W_b.reshape(1, H)
    W2b = W2_b.reshape(1, H)
    args = (a2, l1_lw, lb1, l1_rw, l2_lw, lb2, l2_rw, l3_lw, lb3, l3_rw,
            W_w, Wb, W2_w, W2b)
    out1 = _encode(x1, edge_index_v1, *args)
    out1, x2b, ei2b = lax.optimization_barrier((out1, x2, edge_index_v2))
    out2 = _encode(x2b, ei2b, *args)
    return (out1, out2)


# speedup vs baseline: 4.3647x; 1.0336x over previous
import functools

import jax
import jax.numpy as jnp
from jax import lax
from jax.experimental import pallas as pl
from jax.experimental.pallas import tpu as pltpu
from jax.experimental.pallas import tpu_sc as plsc

NC = 2
NS = 16
LN = 16

N = 10000
NPAD = 10240
E = 320000
D = 128
H = 512

G = 256


@functools.cache
def _segsum_kernel(F, CHUNK, NCHUNK, with_count, n_rows, dtype=jnp.float32):
    CPT = CHUNK // NS
    KPC = NCHUNK // NC
    EPT = E // NS
    EB = 2000
    NB = EPT // EB
    GRP = EB // LN
    FV = F // LN

    assert CHUNK % NS == 0 and NCHUNK % NC == 0 and NCHUNK * CHUNK == NPAD
    assert EPT % EB == 0 and EB % LN == 0 and CPT % LN == 0

    mesh = plsc.VectorSubcoreMesh(core_axis_name="c", subcore_axis_name="s")

    out_type = [jax.ShapeDtypeStruct((NPAD, F), dtype)]
    if with_count:
        out_type.append(jax.ShapeDtypeStruct((NPAD, LN), jnp.float32))

    scratch = [
        pltpu.VMEM((EB,), jnp.int32),
        pltpu.VMEM((EB,), jnp.int32),
        pltpu.VMEM((G // 2,), jnp.int32),
        pltpu.VMEM((G // 2,), jnp.int32),
        pltpu.VMEM((G // 2,), jnp.int32),
        pltpu.VMEM((G // 2,), jnp.int32),
        pltpu.VMEM((G // 2, F), dtype),
        pltpu.VMEM((G // 2, F), dtype),
        pltpu.VMEM((LN, F), dtype),
        pltpu.VMEM_SHARED((CHUNK + LN, F), dtype),
        pltpu.SemaphoreType.DMA,
        pltpu.SemaphoreType.DMA,
    ]
    if with_count:
        scratch += [
            pltpu.VMEM((G // 2, LN), jnp.float32),
            pltpu.VMEM((LN, LN), jnp.float32),
            pltpu.VMEM_SHARED((CHUNK + LN, LN), jnp.float32),
        ]

    def body(src_h, dst_h, x_h, out_h, *rest):
        if with_count:
            (cnt_h, src_v, dst_v, gsrcA, gsrcB, gdstA, gdstB, rowsA, rowsB,
             zbuf, acc, sem, sem2, ones, zbufc, cacc) = rest
        else:
            (src_v, dst_v, gsrcA, gsrcB, gdstA, gdstB, rowsA, rowsB,
             zbuf, acc, sem, sem2) = rest
        GH = G // 2

        c = lax.axis_index("c")
        s = lax.axis_index("s")
        zero16 = jnp.zeros((LN,), jnp.float32)
        izero = jnp.zeros((LN,), jnp.int32)
        pad_src = izero + s
        pad_dst = izero + (CHUNK + s)

        ZW = LN if dtype == jnp.float32 else 2 * LN
        zvec = jnp.zeros((ZW,), dtype)

        def zb(k, _):
            zbuf[k // (F // ZW), pl.ds((k % (F // ZW)) * ZW, ZW)] = zvec
            return 0
        lax.fori_loop(0, LN * (F // ZW), zb, 0)
        if with_count:
            one16 = zero16 + 1.0

            def ob(k, _):
                ones[k, :] = one16
                return 0
            lax.fori_loop(0, G // 2, ob, 0)

            def zc(k, _):
                zbufc[k, :] = zero16
                return 0
            lax.fori_loop(0, LN, zc, 0)

        def refill_pad():
            for j in range(GH // LN):
                gsrcA[pl.ds(j * LN, LN)] = pad_src
                gsrcB[pl.ds(j * LN, LN)] = pad_src
                gdstA[pl.ds(j * LN, LN)] = pad_dst
                gdstB[pl.ds(j * LN, LN)] = pad_dst

        def flush():
            pltpu.async_copy(x_h.at[gsrcA], rowsA, sem).wait()
            sa = pltpu.async_copy(rowsA, acc.at[gdstA], sem2, add=True)
            gb = pltpu.async_copy(x_h.at[gsrcB], rowsB, sem)
            sa.wait()
            gb.wait()
            pltpu.sync_copy(rowsB, acc.at[gdstB], add=True)
            if with_count:
                pltpu.sync_copy(ones, cacc.at[gdstA], add=True)
                pltpu.sync_copy(ones, cacc.at[gdstB], add=True)
            refill_pad()

        for k in range(KPC):
            chunk_id = k * NC + c
            lo = chunk_id * CHUNK

            for i in range(CPT // LN):
                pltpu.sync_copy(zbuf, acc.at[pl.ds(s * CPT + i * LN, LN)])
            if with_count:
                for i in range(CPT // LN):
                    pltpu.sync_copy(zbufc, cacc.at[pl.ds(s * CPT + i * LN, LN)])
            @pl.when(s == 0)
            def _():
                pltpu.sync_copy(zbuf, acc.at[pl.ds(CHUNK, LN)])
                if with_count:
                    pltpu.sync_copy(zbufc, cacc.at[pl.ds(CHUNK, LN)])
            refill_pad()
            plsc.subcore_barrier()

            ebase = s * EPT

            def grp(gi, pend):
                off = gi * LN
                sv = src_v[pl.ds(off, LN)]
                dv = dst_v[pl.ds(off, LN)]
                m = (dv >= lo) & (dv < lo + CHUNK)
                mi = m.astype(jnp.int32)
                pos = pend + plsc.cumsum(mi) - 1
                doff = dv - lo
                mA = m & (pos < GH)
                mB = m & (pos >= GH)
                posB = pos - GH
                plsc.store_scatter(gsrcA, [pos], sv, mask=mA)
                plsc.store_scatter(gdstA, [pos], doff, mask=mA)
                plsc.store_scatter(gsrcB, [posB], sv, mask=mB)
                plsc.store_scatter(gdstB, [posB], doff, mask=mB)
                pend = pend + jnp.sum(mi)
                full = pend > (G - LN)

                @pl.when(full)
                def _():
                    flush()
                return jnp.where(full, 0, pend)

            def blk(b, pend):
                pltpu.sync_copy(src_h.at[pl.ds(ebase + b * EB, EB)], src_v)
                pltpu.sync_copy(dst_h.at[pl.ds(ebase + b * EB, EB)], dst_v)
                return lax.fori_loop(0, GRP, grp, pend)

            lax.fori_loop(0, NB, blk, jnp.int32(0))
            flush()
            plsc.subcore_barrier()

            pltpu.sync_copy(acc.at[pl.ds(s * CPT, CPT)],
                            out_h.at[pl.ds(lo + s * CPT, CPT)])
            if with_count:
                pltpu.sync_copy(cacc.at[pl.ds(s * CPT, CPT)],
                                cnt_h.at[pl.ds(lo + s * CPT, CPT)])

    return pl.kernel(body, out_type=tuple(out_type), mesh=mesh,
                     scratch_types=scratch,
                     compiler_params=pltpu.CompilerParams(
                         needs_layout_passes=False,
                         use_tc_tiling_on_sc=False))


_CT = (((1,), (1,)), ((), ()))


def _dot(a, b):
    return lax.dot_general(a, b, _CT, preferred_element_type=jnp.float32)


@functools.cache
def _layer1_kernel(F_in):
    BR = 256

    def body(s_ref, c_ref, x_ref, a_ref, lw_ref, lb_ref, rw_ref,
             ww_ref, wb_ref, w2_ref, w2b_ref, in2_ref, r3_ref):
        cnt = jnp.maximum(c_ref[:, 0:1], 1.0)
        mean = s_ref[...].astype(jnp.float32) / cnt
        t = _dot(mean, lw_ref[...]) + lb_ref[...] + _dot(x_ref[...], rw_ref[...])
        a = a_ref[...]
        h1 = jnp.where(t >= 0, t, a * t)
        in2_ref[...] = h1 + _dot(x_ref[...], ww_ref[...]) + wb_ref[...]
        r3_ref[...] = h1 + _dot(x_ref[...], w2_ref[...]) + w2b_ref[...]

    grid = (NPAD // BR,)
    row = lambda i: (i, 0)
    fix = lambda i: (0, 0)
    return pl.pallas_call(
        body,
        grid=grid,
        in_specs=[
            pl.BlockSpec((BR, F_in), row),
            pl.BlockSpec((BR, LN), row),
            pl.BlockSpec((BR, F_in), row),
            pl.BlockSpec((1, 1), fix),
            pl.BlockSpec((H, F_in), fix),
            pl.BlockSpec((1, H), fix),
            pl.BlockSpec((H, F_in), fix),
            pl.BlockSpec((H, F_in), fix),
            pl.BlockSpec((1, H), fix),
            pl.BlockSpec((H, F_in), fix),
            pl.BlockSpec((1, H), fix),
        ],
        out_specs=[pl.BlockSpec((BR, H), row), pl.BlockSpec((BR, H), row)],
        out_shape=[jax.ShapeDtypeStruct((NPAD, H), jnp.float32)] * 2,
    )


@functools.cache
def _layer2_kernel():
    BR = 256

    def body(s_ref, c_ref, in_ref, r3_ref, a_ref, lw_ref, lb_ref, rw_ref,
             out_ref):
        cnt = jnp.maximum(c_ref[:, 0:1], 1.0)
        mean = s_ref[...].astype(jnp.float32) / cnt
        t = _dot(mean, lw_ref[...]) + lb_ref[...] + _dot(in_ref[...], rw_ref[...])
        a = a_ref[...]
        h2 = jnp.where(t >= 0, t, a * t)
        out_ref[...] = h2 + r3_ref[...]

    grid = (NPAD // 256,)
    row = lambda i: (i, 0)
    fix = lambda i: (0, 0)
    return pl.pallas_call(
        body,
        grid=grid,
        in_specs=[
            pl.BlockSpec((BR, H), row),
            pl.BlockSpec((BR, LN), row),
            pl.BlockSpec((BR, H), row),
            pl.BlockSpec((BR, H), row),
            pl.BlockSpec((1, 1), fix),
            pl.BlockSpec((H, H), fix),
            pl.BlockSpec((1, H), fix),
            pl.BlockSpec((H, H), fix),
        ],
        out_specs=pl.BlockSpec((BR, H), row),
        out_shape=jax.ShapeDtypeStruct((NPAD, H), jnp.float32),
    )


@functools.cache
def _layer3_kernel():
    BR = 256

    def body(s_ref, c_ref, in_ref, a_ref, lw_ref, lb_ref, rw_ref, out_ref):
        cnt = jnp.maximum(c_ref[:, 0:1], 1.0)
        mean = s_ref[...].astype(jnp.float32) / cnt
        t = _dot(mean, lw_ref[...]) + lb_ref[...] + _dot(in_ref[...], rw_ref[...])
        a = a_ref[...]
        out_ref[...] = jnp.where(t >= 0, t, a * t)

    grid = (NPAD // 256,)
    row = lambda i: (i, 0)
    fix = lambda i: (0, 0)
    return pl.pallas_call(
        body,
        grid=grid,
        in_specs=[
            pl.BlockSpec((BR, H), row),
            pl.BlockSpec((BR, LN), row),
            pl.BlockSpec((BR, H), row),
            pl.BlockSpec((1, 1), fix),
            pl.BlockSpec((H, H), fix),
            pl.BlockSpec((1, H), fix),
            pl.BlockSpec((H, H), fix),
        ],
        out_specs=pl.BlockSpec((BR, H), row),
        out_shape=jax.ShapeDtypeStruct((NPAD, H), jnp.float32),
    )


def _encode(x, ei, a2, lw1, lb1, rw1, lw2, lb2, rw2, lw3, lb3, rw3,
            Ww, Wb, W2w, W2b):
    src = ei[0]
    dst = ei[1]
    seg128 = _segsum_kernel(D, 5120, 2, True, N, jnp.bfloat16)
    seg512 = _segsum_kernel(H, 2560, 4, False, N, jnp.bfloat16)

    s1, c16 = seg128(src, dst, x.astype(jnp.bfloat16))
    xp = jnp.pad(x, ((0, NPAD - N), (0, 0)))
    in2, r3 = _layer1_kernel(D)(s1, c16, xp, a2, lw1, lb1, rw1,
                                Ww, Wb, W2w, W2b)
    (s2,) = seg512(src, dst, in2.astype(jnp.bfloat16))
    in3 = _layer2_kernel()(s2, c16, in2, r3, a2, lw2, lb2, rw2)
    (s3,) = seg512(src, dst, in3.astype(jnp.bfloat16))
    h3 = _layer3_kernel()(s3, c16, in3, a2, lw3, lb3, rw3)
    return h3[:N]


def kernel(x1, x2, edge_index_v1, edge_index_v2, prelu_a, W_w, W_b, W2_w,
           W2_b, l1_lw, l1_lb, l1_rw, l2_lw, l2_lb, l2_rw, l3_lw, l3_lb,
           l3_rw):
    a2 = prelu_a.reshape(1, 1)
    lb1 = l1_lb.reshape(1, H)
    lb2 = l2_lb.reshape(1, H)
    lb3 = l3_lb.reshape(1, H)
    Wb = W_b.reshape(1, H)
    W2b = W2_b.reshape(1, H)
    args = (a2, l1_lw, lb1, l1_rw, l2_lw, lb2, l2_rw, l3_lw, lb3, l3_rw,
            W_w, Wb, W2_w, W2b)
    out1 = _encode(x1, edge_index_v1, *args)
    out1, x2b, ei2b = lax.optimization_barrier((out1, x2, edge_index_v2))
    out2 = _encode(x2b, ei2b, *args)
    return (out1, out2)


# speedup vs baseline: 4.7976x; 1.0992x over previous
import functools

import jax
import jax.numpy as jnp
from jax import lax
from jax.experimental import pallas as pl
from jax.experimental.pallas import tpu as pltpu
from jax.experimental.pallas import tpu_sc as plsc

NC = 2
NS = 16
LN = 16

N = 10000
NPAD = 10240
E = 320000
D = 128
H = 512

G = 256


@functools.cache
def _segsum_kernel(F, CHUNK, NCHUNK, with_count, n_rows, dtype=jnp.float32):
    CPT = CHUNK // NS
    KPC = NCHUNK // NC
    EPT = E // NS
    EB = 2000
    NB = EPT // EB
    GRP = EB // LN
    FV = F // LN

    assert CHUNK % NS == 0 and NCHUNK % NC == 0 and NCHUNK * CHUNK == NPAD
    assert EPT % EB == 0 and EB % LN == 0 and CPT % LN == 0

    mesh = plsc.VectorSubcoreMesh(core_axis_name="c", subcore_axis_name="s")

    out_type = [jax.ShapeDtypeStruct((NPAD, F), dtype)]
    if with_count:
        out_type.append(jax.ShapeDtypeStruct((NPAD, LN), jnp.float32))

    scratch = [
        pltpu.VMEM((EB,), jnp.int32),
        pltpu.VMEM((EB,), jnp.int32),
        pltpu.VMEM((G // 2,), jnp.int32),
        pltpu.VMEM((G // 2,), jnp.int32),
        pltpu.VMEM((G // 2,), jnp.int32),
        pltpu.VMEM((G // 2,), jnp.int32),
        pltpu.VMEM((G // 2, F), dtype),
        pltpu.VMEM((G // 2, F), dtype),
        pltpu.VMEM((LN, F), dtype),
        pltpu.VMEM_SHARED((CHUNK + LN, F), dtype),
        pltpu.SemaphoreType.DMA,
        pltpu.SemaphoreType.DMA,
    ]
    if with_count:
        scratch += [
            pltpu.VMEM((G // 2, LN), jnp.float32),
            pltpu.VMEM((LN, LN), jnp.float32),
            pltpu.VMEM_SHARED((CHUNK + LN, LN), jnp.float32),
        ]

    def body(src_h, dst_h, x_h, out_h, *rest):
        if with_count:
            (cnt_h, src_v, dst_v, gsrcA, gsrcB, gdstA, gdstB, rowsA, rowsB,
             zbuf, acc, sem, sem2, ones, zbufc, cacc) = rest
        else:
            (src_v, dst_v, gsrcA, gsrcB, gdstA, gdstB, rowsA, rowsB,
             zbuf, acc, sem, sem2) = rest
        GH = G // 2

        c = lax.axis_index("c")
        s = lax.axis_index("s")
        zero16 = jnp.zeros((LN,), jnp.float32)
        izero = jnp.zeros((LN,), jnp.int32)
        pad_src = izero + s
        pad_dst = izero + (CHUNK + s)

        ZW = LN if dtype == jnp.float32 else 2 * LN
        zvec = jnp.zeros((ZW,), dtype)

        def zb(k, _):
            zbuf[k // (F // ZW), pl.ds((k % (F // ZW)) * ZW, ZW)] = zvec
            return 0
        lax.fori_loop(0, LN * (F // ZW), zb, 0)
        if with_count:
            one16 = zero16 + 1.0

            def ob(k, _):
                ones[k, :] = one16
                return 0
            lax.fori_loop(0, G // 2, ob, 0)

            def zc(k, _):
                zbufc[k, :] = zero16
                return 0
            lax.fori_loop(0, LN, zc, 0)

        def refill_pad():
            for j in range(GH // LN):
                gsrcA[pl.ds(j * LN, LN)] = pad_src
                gsrcB[pl.ds(j * LN, LN)] = pad_src
                gdstA[pl.ds(j * LN, LN)] = pad_dst
                gdstB[pl.ds(j * LN, LN)] = pad_dst

        def flush():
            pltpu.async_copy(x_h.at[gsrcA], rowsA, sem).wait()
            sa = pltpu.async_copy(rowsA, acc.at[gdstA], sem2, add=True)
            gb = pltpu.async_copy(x_h.at[gsrcB], rowsB, sem)
            sa.wait()
            gb.wait()
            pltpu.sync_copy(rowsB, acc.at[gdstB], add=True)
            if with_count:
                pltpu.sync_copy(ones, cacc.at[gdstA], add=True)
                pltpu.sync_copy(ones, cacc.at[gdstB], add=True)
            refill_pad()

        for k in range(KPC):
            chunk_id = k * NC + c
            lo = chunk_id * CHUNK

            for i in range(CPT // LN):
                pltpu.sync_copy(zbuf, acc.at[pl.ds(s * CPT + i * LN, LN)])
            if with_count:
                for i in range(CPT // LN):
                    pltpu.sync_copy(zbufc, cacc.at[pl.ds(s * CPT + i * LN, LN)])
            @pl.when(s == 0)
            def _():
                pltpu.sync_copy(zbuf, acc.at[pl.ds(CHUNK, LN)])
                if with_count:
                    pltpu.sync_copy(zbufc, cacc.at[pl.ds(CHUNK, LN)])
            refill_pad()
            plsc.subcore_barrier()

            ebase = s * EPT

            def grp(gi, pend):
                off = gi * LN
                sv = src_v[pl.ds(off, LN)]
                dv = dst_v[pl.ds(off, LN)]
                m = (dv >= lo) & (dv < lo + CHUNK)
                mi = m.astype(jnp.int32)
                pos = pend + plsc.cumsum(mi) - 1
                doff = dv - lo
                mA = m & (pos < GH)
                mB = m & (pos >= GH)
                posB = pos - GH
                plsc.store_scatter(gsrcA, [pos], sv, mask=mA)
                plsc.store_scatter(gdstA, [pos], doff, mask=mA)
                plsc.store_scatter(gsrcB, [posB], sv, mask=mB)
                plsc.store_scatter(gdstB, [posB], doff, mask=mB)
                pend = pend + jnp.sum(mi)
                full = pend > (G - LN)

                @pl.when(full)
                def _():
                    flush()
                return jnp.where(full, 0, pend)

            def blk(b, pend):
                pltpu.sync_copy(src_h.at[pl.ds(ebase + b * EB, EB)], src_v)
                pltpu.sync_copy(dst_h.at[pl.ds(ebase + b * EB, EB)], dst_v)
                return lax.fori_loop(0, GRP, grp, pend)

            lax.fori_loop(0, NB, blk, jnp.int32(0))
            flush()
            plsc.subcore_barrier()

            pltpu.sync_copy(acc.at[pl.ds(s * CPT, CPT)],
                            out_h.at[pl.ds(lo + s * CPT, CPT)])
            if with_count:
                pltpu.sync_copy(cacc.at[pl.ds(s * CPT, CPT)],
                                cnt_h.at[pl.ds(lo + s * CPT, CPT)])

    return pl.kernel(body, out_type=tuple(out_type), mesh=mesh,
                     scratch_types=scratch,
                     compiler_params=pltpu.CompilerParams(
                         needs_layout_passes=False,
                         use_tc_tiling_on_sc=False))


_CT = (((1,), (1,)), ((), ()))


def _dot(a, b):
    return lax.dot_general(a, b, _CT, preferred_element_type=jnp.float32)


@functools.cache
def _layer1_kernel(F_in):
    BR = 256

    def body(s_ref, c_ref, x_ref, a_ref, lw_ref, lb_ref, rw_ref,
             ww_ref, wb_ref, w2_ref, w2b_ref, in2_ref, r3_ref):
        cnt = jnp.maximum(c_ref[:, 0:1], 1.0)
        mean = s_ref[...].astype(jnp.float32) / cnt
        t = _dot(mean, lw_ref[...]) + lb_ref[...] + _dot(x_ref[...], rw_ref[...])
        a = a_ref[...]
        h1 = jnp.where(t >= 0, t, a * t)
        in2_ref[...] = h1 + _dot(x_ref[...], ww_ref[...]) + wb_ref[...]
        r3_ref[...] = h1 + _dot(x_ref[...], w2_ref[...]) + w2b_ref[...]

    grid = (NPAD // BR,)
    row = lambda i: (i, 0)
    fix = lambda i: (0, 0)
    return pl.pallas_call(
        body,
        grid=grid,
        in_specs=[
            pl.BlockSpec((BR, F_in), row),
            pl.BlockSpec((BR, LN), row),
            pl.BlockSpec((BR, F_in), row),
            pl.BlockSpec((1, 1), fix),
            pl.BlockSpec((H, F_in), fix),
            pl.BlockSpec((1, H), fix),
            pl.BlockSpec((H, F_in), fix),
            pl.BlockSpec((H, F_in), fix),
            pl.BlockSpec((1, H), fix),
            pl.BlockSpec((H, F_in), fix),
            pl.BlockSpec((1, H), fix),
        ],
        out_specs=[pl.BlockSpec((BR, H), row), pl.BlockSpec((BR, H), row)],
        out_shape=[jax.ShapeDtypeStruct((NPAD, H), jnp.float32)] * 2,
    )


@functools.cache
def _layer2_kernel():
    BR = 256

    def body(s_ref, c_ref, in_ref, r3_ref, a_ref, lw_ref, lb_ref, rw_ref,
             out_ref):
        cnt = jnp.maximum(c_ref[:, 0:1], 1.0)
        mean = s_ref[...].astype(jnp.float32) / cnt
        t = _dot(mean, lw_ref[...]) + lb_ref[...] + _dot(in_ref[...], rw_ref[...])
        a = a_ref[...]
        h2 = jnp.where(t >= 0, t, a * t)
        out_ref[...] = h2 + r3_ref[...]

    grid = (NPAD // 256,)
    row = lambda i: (i, 0)
    fix = lambda i: (0, 0)
    return pl.pallas_call(
        body,
        grid=grid,
        in_specs=[
            pl.BlockSpec((BR, H), row),
            pl.BlockSpec((BR, LN), row),
            pl.BlockSpec((BR, H), row),
            pl.BlockSpec((BR, H), row),
            pl.BlockSpec((1, 1), fix),
            pl.BlockSpec((H, H), fix),
            pl.BlockSpec((1, H), fix),
            pl.BlockSpec((H, H), fix),
        ],
        out_specs=pl.BlockSpec((BR, H), row),
        out_shape=jax.ShapeDtypeStruct((NPAD, H), jnp.float32),
    )


@functools.cache
def _layer3_kernel():
    BR = 256

    def body(s_ref, c_ref, in_ref, a_ref, lw_ref, lb_ref, rw_ref, out_ref):
        cnt = jnp.maximum(c_ref[:, 0:1], 1.0)
        mean = s_ref[...].astype(jnp.float32) / cnt
        t = _dot(mean, lw_ref[...]) + lb_ref[...] + _dot(in_ref[...], rw_ref[...])
        a = a_ref[...]
        out_ref[...] = jnp.where(t >= 0, t, a * t)

    grid = (NPAD // 256,)
    row = lambda i: (i, 0)
    fix = lambda i: (0, 0)
    return pl.pallas_call(
        body,
        grid=grid,
        in_specs=[
            pl.BlockSpec((BR, H), row),
            pl.BlockSpec((BR, LN), row),
            pl.BlockSpec((BR, H), row),
            pl.BlockSpec((1, 1), fix),
            pl.BlockSpec((H, H), fix),
            pl.BlockSpec((1, H), fix),
            pl.BlockSpec((H, H), fix),
        ],
        out_specs=pl.BlockSpec((BR, H), row),
        out_shape=jax.ShapeDtypeStruct((NPAD, H), jnp.float32),
    )


def _encode2(xa, eia, xb, eib, a2, lw1, lb1, rw1, lw2, lb2, rw2,
             lw3, lb3, rw3, Ww, Wb, W2w, W2b):
    seg128 = _segsum_kernel(D, 5120, 2, True, N, jnp.bfloat16)
    seg512 = _segsum_kernel(H, 2560, 4, False, N, jnp.bfloat16)
    sa, da = eia[0], eia[1]
    sb, db = eib[0], eib[1]
    l1 = _layer1_kernel(D)
    l2 = _layer2_kernel()
    l3 = _layer3_kernel()

    s1a, c16a = seg128(sa, da, xa.astype(jnp.bfloat16))
    s1b, c16b = seg128(sb, db, xb.astype(jnp.bfloat16))
    xpa = jnp.pad(xa, ((0, NPAD - N), (0, 0)))
    xpb = jnp.pad(xb, ((0, NPAD - N), (0, 0)))
    in2a, r3a = l1(s1a, c16a, xpa, a2, lw1, lb1, rw1, Ww, Wb, W2w, W2b)
    (s2a,) = seg512(sa, da, in2a.astype(jnp.bfloat16))
    in2b, r3b = l1(s1b, c16b, xpb, a2, lw1, lb1, rw1, Ww, Wb, W2w, W2b)
    (s2b,) = seg512(sb, db, in2b.astype(jnp.bfloat16))
    in3a = l2(s2a, c16a, in2a, r3a, a2, lw2, lb2, rw2)
    (s3a,) = seg512(sa, da, in3a.astype(jnp.bfloat16))
    in3b = l2(s2b, c16b, in2b, r3b, a2, lw2, lb2, rw2)
    (s3b,) = seg512(sb, db, in3b.astype(jnp.bfloat16))
    h3a = l3(s3a, c16a, in3a, a2, lw3, lb3, rw3)
    h3b = l3(s3b, c16b, in3b, a2, lw3, lb3, rw3)
    return h3a[:N], h3b[:N]


def kernel(x1, x2, edge_index_v1, edge_index_v2, prelu_a, W_w, W_b, W2_w,
           W2_b, l1_lw, l1_lb, l1_rw, l2_lw, l2_lb, l2_rw, l3_lw, l3_lb,
           l3_rw):
    a2 = prelu_a.reshape(1, 1)
    lb1 = l1_lb.reshape(1, H)
    lb2 = l2_lb.reshape(1, H)
    lb3 = l3_lb.reshape(1, H)
    Wb = W_b.reshape(1, H)
    W2b = W2_b.reshape(1, H)
    args = (a2, l1_lw, lb1, l1_rw, l2_lw, lb2, l2_rw, l3_lw, lb3, l3_rw,
            W_w, Wb, W2_w, W2b)
    out1, out2 = _encode2(x1, edge_index_v1, x2, edge_index_v2, *args)
    return (out1, out2)
